# Initial kernel scaffold; baseline (speedup 1.0000x reference)
import jax
import jax.numpy as jnp
from jax.experimental import pallas as pl


def kernel(h, edge_index, edge_attr, params):
    raise NotImplementedError("write your pallas kernel here")


# speedup vs baseline: 3.1511x; 3.1511x over previous
import functools

import jax
import jax.numpy as jnp
from jax import lax
from jax.experimental import pallas as pl
from jax.experimental.pallas import tpu as pltpu
from jax.experimental.pallas import tpu_sc as plsc

NN = 10000
NP = 10240
RW = 144
EE = 320000
NC, NS = 2, 16
NW = NC * NS
EPT = EE // NW
CH = 80
NCHK = EPT // CH
RPS = NP // NS
DH = 256
NBLK = 8
BLK = NP // NBLK
EBLK = 3200
NEB = EE // EBLK
EPS = 1e-5


def _ln(x, g, b):
    m = jnp.mean(x, axis=1, keepdims=True)
    v = jnp.mean((x - m) ** 2, axis=1, keepdims=True)
    return (x - m) * lax.rsqrt(v + EPS) * g + b


def _proj_body(h_ref, w0, b0, g0, be0, w1, b1, g1, be1, out_ref):
    h = h_ref[...]
    z0 = jnp.maximum(_ln(h[:, :64] @ w0[...] + b0[...], g0[...], be0[...]), 0.0)
    z1 = jnp.maximum(_ln(h[:, 64:] @ w1[...] + b1[...], g1[...], be1[...]), 0.0)
    out_ref[:, 0:64] = z0
    out_ref[:, 64:128] = z1
    col = lax.broadcasted_iota(jnp.int32, (BLK, 16), 1)
    out_ref[:, 128:144] = jnp.where(col == 0, 1.0, 0.0)


def _sc_scatter_body(aug_hbm, srcp_hbm, dstp_hbm, zr_hbm, acc_hbm,
                     acc_sh, src_v, dst_v, rows_v, sem):
    c = lax.axis_index("c")
    s = lax.axis_index("s")
    wid = s * NC + c
    pltpu.sync_copy(zr_hbm, acc_sh.at[pl.ds(s * RPS, RPS)])
    plsc.subcore_barrier()
    pltpu.sync_copy(srcp_hbm.at[wid], src_v)
    pltpu.sync_copy(dstp_hbm.at[wid], dst_v)

    def chunk(j, carry):
        pltpu.async_copy(aug_hbm.at[src_v.at[j]], rows_v, sem).wait()
        pltpu.sync_copy(rows_v, acc_sh.at[dst_v.at[j]], add=True)
        return carry

    lax.fori_loop(0, NCHK, chunk, 0)
    plsc.subcore_barrier()
    pltpu.sync_copy(acc_sh.at[pl.ds(s * RPS, RPS)],
                    acc_hbm.at[c, pl.ds(s * RPS, RPS)])


def _mid_body(aug_ref, acc_ref, gwa, gwb, gb, gg, gbe,
              nwp, nbp, ngp, nbep, w1a, w1bp, w1d, w1ep, b1,
              n_ref, a_ref, b_ref):
    hp = aug_ref[:, :128]
    ssum = acc_ref[0] + acc_ref[1]
    ah = ssum[:, :128]
    deg = ssum[:, 128:129]
    nrm = jnp.where(deg > 0, 1.0 / deg, 0.0)
    pre = hp @ gwa[...] + (ah * nrm) @ gwb[...] + gb[...]
    hg = jnp.maximum(_ln(pre, gg[...], gbe[...]), 0.0)
    nh = hg @ nwp[...] + nbp[...]
    cmask = (lax.broadcasted_iota(jnp.int32, (1, 128), 1) < 5).astype(jnp.float32)
    nm = jnp.sum(nh * cmask, axis=1, keepdims=True) * (1.0 / 5.0)
    d = (nh - nm) * cmask
    nv = jnp.sum(d * d, axis=1, keepdims=True) * (1.0 / 5.0)
    nf = d * lax.rsqrt(nv + EPS) * ngp[...] + nbep[...]
    n_ref[...] = nf
    a_ref[...] = hg @ w1a[...] + nf @ w1bp[...] + b1[...]
    b_ref[...] = hg @ w1d[...] + nf @ w1ep[...]


def _sc_gather_body(a_hbm, b_hbm, srcp_hbm, dstp_hbm, out_hbm,
                    src_v, dst_v, bufa, bufb, sema, semb):
    c = lax.axis_index("c")
    s = lax.axis_index("s")
    wid = s * NC + c
    pltpu.sync_copy(srcp_hbm.at[wid], src_v)
    pltpu.sync_copy(dstp_hbm.at[wid], dst_v)
    base = wid * EPT

    def chunk(j, carry):
        cpa = pltpu.async_copy(a_hbm.at[src_v.at[j]], bufa, sema)
        cpb = pltpu.async_copy(b_hbm.at[dst_v.at[j]], bufb, semb)
        cpa.wait()
        cpb.wait()

        def row(r, cc):
            for k in range(DH // 16):
                sl = pl.ds(k * 16, 16)
                bufa[r, sl] = bufa[r, sl] + bufb[r, sl]
            return cc

        lax.fori_loop(0, CH, row, 0)
        pltpu.sync_copy(bufa, out_hbm.at[pl.ds(base + j * CH, CH)])
        return carry

    lax.fori_loop(0, NCHK, chunk, 0)


def _edge_body(t_ref, ea_ref, w1cp, lg, lb, w2p, b2p, out_ref):
    x = t_ref[...] + ea_ref[...] @ w1cp[...]
    y = jnp.maximum(_ln(x, lg[...], lb[...]), 0.0)
    e = y @ w2p[...] + b2p[...]
    out_ref[...] = e[:, :2]


def _full(shape):
    return pl.BlockSpec(shape, lambda i: tuple(0 for _ in shape))


def kernel(h, edge_index, edge_attr, params):
    f32 = jnp.float32
    hpad = jnp.zeros((NP, 128), f32).at[:NN].set(h)
    srcp = edge_index[0].reshape(NW, NCHK, CH)
    dstp = edge_index[1].reshape(NW, NCHK, CH)
    eap = jnp.zeros((EE, 8), f32).at[:, :4].set(edge_attr)
    zr = jnp.zeros((RPS, RW), f32)

    p0, p1 = params["proj"][0], params["proj"][1]
    row = lambda v: v.reshape(1, -1)

    aug = pl.pallas_call(
        _proj_body,
        grid=(NBLK,),
        in_specs=[pl.BlockSpec((BLK, 128), lambda i: (i, 0)),
                  _full((64, 64)), _full((1, 64)), _full((1, 64)), _full((1, 64)),
                  _full((64, 64)), _full((1, 64)), _full((1, 64)), _full((1, 64))],
        out_specs=pl.BlockSpec((BLK, RW), lambda i: (i, 0)),
        out_shape=jax.ShapeDtypeStruct((NP, RW), f32),
    )(hpad, p0["W"], row(p0["b"]), row(p0["g"]), row(p0["beta"]),
      p1["W"], row(p1["b"]), row(p1["g"]), row(p1["beta"]))

    mesh = plsc.VectorSubcoreMesh(core_axis_name="c", subcore_axis_name="s",
                                  num_cores=NC, num_subcores=NS)
    scatter = pl.kernel(
        _sc_scatter_body,
        out_type=jax.ShapeDtypeStruct((NC, NP, RW), f32),
        mesh=mesh,
        compiler_params=pltpu.CompilerParams(use_tc_tiling_on_sc=False),
        scratch_types=[
            pltpu.VMEM_SHARED((NP, RW), f32),
            pltpu.VMEM((NCHK, CH), jnp.int32),
            pltpu.VMEM((NCHK, CH), jnp.int32),
            pltpu.VMEM((CH, RW), f32),
            pltpu.SemaphoreType.DMA,
        ],
    )
    acc = scatter(aug, srcp, dstp, zr)

    W1 = params["W1"]
    nWp = jnp.zeros((128, 128), f32).at[:, :5].set(params["node_W"])
    nbp = jnp.zeros((1, 128), f32).at[0, :5].set(params["node_b"])
    ngp = jnp.zeros((1, 128), f32).at[0, :5].set(params["node_g"])
    nbep = jnp.zeros((1, 128), f32).at[0, :5].set(params["node_beta"])
    W1bp = jnp.zeros((128, DH), f32).at[:5].set(W1[128:133])
    W1ep = jnp.zeros((128, DH), f32).at[:5].set(W1[265:270])
    nfull, tabA, tabB = pl.pallas_call(
        _mid_body,
        grid=(NBLK,),
        in_specs=[pl.BlockSpec((BLK, RW), lambda i: (i, 0)),
                  pl.BlockSpec((NC, BLK, RW), lambda i: (0, i, 0)),
                  _full((128, 128)), _full((128, 128)), _full((1, 128)),
                  _full((1, 128)), _full((1, 128)),
                  _full((128, 128)), _full((1, 128)), _full((1, 128)), _full((1, 128)),
                  _full((128, DH)), _full((128, DH)), _full((128, DH)),
                  _full((128, DH)), _full((1, DH))],
        out_specs=[pl.BlockSpec((BLK, 128), lambda i: (i, 0)),
                   pl.BlockSpec((BLK, DH), lambda i: (i, 0)),
                   pl.BlockSpec((BLK, DH), lambda i: (i, 0))],
        out_shape=[jax.ShapeDtypeStruct((NP, 128), f32),
                   jax.ShapeDtypeStruct((NP, DH), f32),
                   jax.ShapeDtypeStruct((NP, DH), f32)],
    )(aug, acc,
      params["gcn_W"][:128], params["gcn_W"][128:], row(params["gcn_b"]),
      row(params["gcn_g"]), row(params["gcn_beta"]),
      nWp, nbp, ngp, nbep,
      W1[0:128], W1bp, W1[137:265], W1ep, row(params["b1"]))

    gather = pl.kernel(
        _sc_gather_body,
        out_type=jax.ShapeDtypeStruct((EE, DH), f32),
        mesh=mesh,
        compiler_params=pltpu.CompilerParams(use_tc_tiling_on_sc=False),
        scratch_types=[
            pltpu.VMEM((NCHK, CH), jnp.int32),
            pltpu.VMEM((NCHK, CH), jnp.int32),
            pltpu.VMEM((CH, DH), f32),
            pltpu.VMEM((CH, DH), f32),
            pltpu.SemaphoreType.DMA,
            pltpu.SemaphoreType.DMA,
        ],
    )
    tsum = gather(tabA, tabB, srcp, dstp)

    W2p = jnp.zeros((DH, 128), f32).at[:, :2].set(params["W2"])
    b2p = jnp.zeros((1, 128), f32).at[0, :2].set(params["b2"])
    e = pl.pallas_call(
        _edge_body,
        grid=(NEB,),
        in_specs=[pl.BlockSpec((EBLK, DH), lambda i: (i, 0)),
                  pl.BlockSpec((EBLK, 8), lambda i: (i, 0)),
                  _full((8, DH)), _full((1, DH)), _full((1, DH)),
                  _full((DH, 128)), _full((1, 128))],
        out_specs=pl.BlockSpec((EBLK, 2), lambda i: (i, 0)),
        out_shape=jax.ShapeDtypeStruct((EE, 2), f32),
    )(tsum, eap, jnp.zeros((8, DH), f32).at[:4].set(W1[133:137]),
      row(params["ln1_g"]), row(params["ln1_b"]), W2p, b2p)

    n = nfull[:NN, :5]
    return (n, e)


# speedup vs baseline: 4.3066x; 1.3667x over previous
import functools

import jax
import jax.numpy as jnp
from jax import lax
from jax.experimental import pallas as pl
from jax.experimental.pallas import tpu as pltpu
from jax.experimental.pallas import tpu_sc as plsc

NN = 10000
NP = 10240
RW = 144
EE = 320000
NC, NS = 2, 16
NW = NC * NS
EPT = EE // NW
CH = 80
NCHK = EPT // CH
RPS = NP // NS
DH = 256
NBLK = 8
BLK = NP // NBLK
EBLK = 3200
NEB = EE // EBLK
EPS = 1e-5


def _ln(x, g, b):
    m = jnp.mean(x, axis=1, keepdims=True)
    v = jnp.mean((x - m) ** 2, axis=1, keepdims=True)
    return (x - m) * lax.rsqrt(v + EPS) * g + b


def _proj_body(h_ref, w0, b0, g0, be0, w1, b1, g1, be1, out_ref):
    h = h_ref[...]
    z0 = jnp.maximum(_ln(h[:, :64] @ w0[...] + b0[...], g0[...], be0[...]), 0.0)
    z1 = jnp.maximum(_ln(h[:, 64:] @ w1[...] + b1[...], g1[...], be1[...]), 0.0)
    out_ref[:, 0:64] = z0
    out_ref[:, 64:128] = z1
    col = lax.broadcasted_iota(jnp.int32, (BLK, 16), 1)
    out_ref[:, 128:144] = jnp.where(col == 0, 1.0, 0.0)


def _sc_scatter_body(aug_hbm, srcp_hbm, dstp_hbm, zr_hbm, acc_hbm,
                     acc_sh, src_v, dst_v, rows_v, sem):
    c = lax.axis_index("c")
    s = lax.axis_index("s")
    wid = s * NC + c
    pltpu.sync_copy(zr_hbm, acc_sh.at[pl.ds(s * RPS, RPS)])
    plsc.subcore_barrier()
    pltpu.sync_copy(srcp_hbm.at[wid], src_v)
    pltpu.sync_copy(dstp_hbm.at[wid], dst_v)

    def chunk(j, carry):
        pltpu.async_copy(aug_hbm.at[src_v.at[j]], rows_v, sem).wait()
        pltpu.sync_copy(rows_v, acc_sh.at[dst_v.at[j]], add=True)
        return carry

    lax.fori_loop(0, NCHK, chunk, 0)
    plsc.subcore_barrier()
    pltpu.sync_copy(acc_sh.at[pl.ds(s * RPS, RPS)],
                    acc_hbm.at[c, pl.ds(s * RPS, RPS)])


def _mid_body(aug_ref, acc_ref, gwa, gwb, gb, gg, gbe,
              nwp, nbp, ngp, nbep, w1a, w1bp, w1d, w1ep, b1,
              n_ref, a_ref, b_ref):
    hp = aug_ref[:, :128]
    ssum = acc_ref[0] + acc_ref[1]
    ah = ssum[:, :128]
    deg = ssum[:, 128:129]
    nrm = jnp.where(deg > 0, 1.0 / deg, 0.0)
    pre = hp @ gwa[...] + (ah * nrm) @ gwb[...] + gb[...]
    hg = jnp.maximum(_ln(pre, gg[...], gbe[...]), 0.0)
    nh = hg @ nwp[...] + nbp[...]
    cmask = (lax.broadcasted_iota(jnp.int32, (1, 128), 1) < 5).astype(jnp.float32)
    nm = jnp.sum(nh * cmask, axis=1, keepdims=True) * (1.0 / 5.0)
    d = (nh - nm) * cmask
    nv = jnp.sum(d * d, axis=1, keepdims=True) * (1.0 / 5.0)
    nf = d * lax.rsqrt(nv + EPS) * ngp[...] + nbep[...]
    n_ref[...] = nf
    a_ref[...] = hg @ w1a[...] + nf @ w1bp[...] + b1[...]
    b_ref[...] = hg @ w1d[...] + nf @ w1ep[...]


def _sc_gather_body(a_hbm, b_hbm, srcp_hbm, dstp_hbm, out_hbm,
                    src_v, dst_v, bufa, bufb, sema, semb):
    c = lax.axis_index("c")
    s = lax.axis_index("s")
    wid = s * NC + c
    pltpu.sync_copy(srcp_hbm.at[wid], src_v)
    pltpu.sync_copy(dstp_hbm.at[wid], dst_v)
    base = wid * EPT

    def chunk(j, carry):
        cpa = pltpu.async_copy(a_hbm.at[src_v.at[j]], bufa, sema)
        cpb = pltpu.async_copy(b_hbm.at[dst_v.at[j]], bufb, semb)
        cpa.wait()
        cpb.wait()

        def row(r, cc):
            for k in range(DH // 16):
                sl = pl.ds(k * 16, 16)
                bufa[r, sl] = bufa[r, sl] + bufb[r, sl]
            return cc

        lax.fori_loop(0, CH, row, 0)
        pltpu.sync_copy(bufa, out_hbm.at[pl.ds(base + j * CH, CH)])
        return carry

    lax.fori_loop(0, NCHK, chunk, 0)


def _edge_body(t_ref, ea_ref, w1cp, lg, lb, w2p, b2p, out_ref):
    x = t_ref[...] + ea_ref[...] @ w1cp[...]
    y = jnp.maximum(_ln(x, lg[...], lb[...]), 0.0)
    e = y @ w2p[...] + b2p[...]
    out_ref[...] = e[:, :2]


def _full(shape):
    return pl.BlockSpec(shape, lambda i: tuple(0 for _ in shape))


def kernel(h, edge_index, edge_attr, params):
    f32 = jnp.float32
    hpad = jnp.zeros((NP, 128), f32).at[:NN].set(h)
    srcp = edge_index[0].reshape(NW, NCHK, CH)
    dstp = edge_index[1].reshape(NW, NCHK, CH)
    zr = jnp.zeros((RPS, RW), f32)

    p0, p1 = params["proj"][0], params["proj"][1]
    row = lambda v: v.reshape(1, -1)

    aug = pl.pallas_call(
        _proj_body,
        grid=(NBLK,),
        in_specs=[pl.BlockSpec((BLK, 128), lambda i: (i, 0)),
                  _full((64, 64)), _full((1, 64)), _full((1, 64)), _full((1, 64)),
                  _full((64, 64)), _full((1, 64)), _full((1, 64)), _full((1, 64))],
        out_specs=pl.BlockSpec((BLK, RW), lambda i: (i, 0)),
        out_shape=jax.ShapeDtypeStruct((NP, RW), f32),
    )(hpad, p0["W"], row(p0["b"]), row(p0["g"]), row(p0["beta"]),
      p1["W"], row(p1["b"]), row(p1["g"]), row(p1["beta"]))

    mesh = plsc.VectorSubcoreMesh(core_axis_name="c", subcore_axis_name="s",
                                  num_cores=NC, num_subcores=NS)
    scatter = pl.kernel(
        _sc_scatter_body,
        out_type=jax.ShapeDtypeStruct((NC, NP, RW), f32),
        mesh=mesh,
        compiler_params=pltpu.CompilerParams(use_tc_tiling_on_sc=False),
        scratch_types=[
            pltpu.VMEM_SHARED((NP, RW), f32),
            pltpu.VMEM((NCHK, CH), jnp.int32),
            pltpu.VMEM((NCHK, CH), jnp.int32),
            pltpu.VMEM((CH, RW), f32),
            pltpu.SemaphoreType.DMA,
        ],
    )
    acc = scatter(aug, srcp, dstp, zr)

    W1 = params["W1"]
    nWp = jnp.zeros((128, 128), f32).at[:, :5].set(params["node_W"])
    nbp = jnp.zeros((1, 128), f32).at[0, :5].set(params["node_b"])
    ngp = jnp.zeros((1, 128), f32).at[0, :5].set(params["node_g"])
    nbep = jnp.zeros((1, 128), f32).at[0, :5].set(params["node_beta"])
    W1bp = jnp.zeros((128, DH), f32).at[:5].set(W1[128:133])
    W1ep = jnp.zeros((128, DH), f32).at[:5].set(W1[265:270])
    nfull, tabA, tabB = pl.pallas_call(
        _mid_body,
        grid=(NBLK,),
        in_specs=[pl.BlockSpec((BLK, RW), lambda i: (i, 0)),
                  pl.BlockSpec((NC, BLK, RW), lambda i: (0, i, 0)),
                  _full((128, 128)), _full((128, 128)), _full((1, 128)),
                  _full((1, 128)), _full((1, 128)),
                  _full((128, 128)), _full((1, 128)), _full((1, 128)), _full((1, 128)),
                  _full((128, DH)), _full((128, DH)), _full((128, DH)),
                  _full((128, DH)), _full((1, DH))],
        out_specs=[pl.BlockSpec((BLK, 128), lambda i: (i, 0)),
                   pl.BlockSpec((BLK, DH), lambda i: (i, 0)),
                   pl.BlockSpec((BLK, DH), lambda i: (i, 0))],
        out_shape=[jax.ShapeDtypeStruct((NP, 128), f32),
                   jax.ShapeDtypeStruct((NP, DH), f32),
                   jax.ShapeDtypeStruct((NP, DH), f32)],
    )(aug, acc,
      params["gcn_W"][:128], params["gcn_W"][128:], row(params["gcn_b"]),
      row(params["gcn_g"]), row(params["gcn_beta"]),
      nWp, nbp, ngp, nbep,
      W1[0:128], W1bp, W1[137:265], W1ep, row(params["b1"]))

    gather = pl.kernel(
        _sc_gather_body,
        out_type=jax.ShapeDtypeStruct((EE, DH), f32),
        mesh=mesh,
        scratch_types=[
            pltpu.VMEM((NCHK, CH), jnp.int32),
            pltpu.VMEM((NCHK, CH), jnp.int32),
            pltpu.VMEM((CH, DH), f32),
            pltpu.VMEM((CH, DH), f32),
            pltpu.SemaphoreType.DMA,
            pltpu.SemaphoreType.DMA,
        ],
    )
    tsum = gather(tabA, tabB, srcp, dstp)

    W2p = jnp.zeros((DH, 128), f32).at[:, :2].set(params["W2"])
    b2p = jnp.zeros((1, 128), f32).at[0, :2].set(params["b2"])
    e = pl.pallas_call(
        _edge_body,
        grid=(NEB,),
        in_specs=[pl.BlockSpec((EBLK, DH), lambda i: (i, 0)),
                  pl.BlockSpec((EBLK, 4), lambda i: (i, 0)),
                  _full((4, DH)), _full((1, DH)), _full((1, DH)),
                  _full((DH, 128)), _full((1, 128))],
        out_specs=pl.BlockSpec((EBLK, 2), lambda i: (i, 0)),
        out_shape=jax.ShapeDtypeStruct((EE, 2), f32),
    )(tsum, edge_attr, W1[133:137],
      row(params["ln1_g"]), row(params["ln1_b"]), W2p, b2p)

    n = nfull[:NN, :5]
    return (n, e)


# speedup vs baseline: 5.5829x; 1.2964x over previous
import functools

import jax
import jax.numpy as jnp
from jax import lax
from jax.experimental import pallas as pl
from jax.experimental.pallas import tpu as pltpu
from jax.experimental.pallas import tpu_sc as plsc

NN = 10000
NP = 10240
RW = 136
EE = 320000
NC, NS = 2, 16
NW = NC * NS
EPT = EE // NW
CH = 80
NCHK = EPT // CH
RPS = NP // NS
DH = 256
NBLK = 8
BLK = NP // NBLK
EBLK = 3200
NEB = EE // EBLK
EPS = 1e-5


def _ln(x, g, b):
    m = jnp.mean(x, axis=1, keepdims=True)
    v = jnp.mean((x - m) ** 2, axis=1, keepdims=True)
    return (x - m) * lax.rsqrt(v + EPS) * g + b


def _proj_body(h_ref, w0, b0, g0, be0, w1, b1, g1, be1, out_ref):
    h = h_ref[...]
    z0 = jnp.maximum(_ln(h[:, :64] @ w0[...] + b0[...], g0[...], be0[...]), 0.0)
    z1 = jnp.maximum(_ln(h[:, 64:] @ w1[...] + b1[...], g1[...], be1[...]), 0.0)
    out_ref[:, 0:64] = z0
    out_ref[:, 64:128] = z1
    col = lax.broadcasted_iota(jnp.int32, (BLK, 8), 1)
    out_ref[:, 128:136] = jnp.where(col == 0, 1.0, 0.0)


def _sc_scatter_body(aug_hbm, srcp_hbm, dstp_hbm, zr_hbm, acc_hbm,
                     acc_sh, src_v, dst_v, rows_v, rows_v1, sem, sem1):
    c = lax.axis_index("c")
    s = lax.axis_index("s")
    wid = s * NC + c
    pltpu.sync_copy(zr_hbm, acc_sh.at[pl.ds(s * RPS, RPS)])
    plsc.subcore_barrier()
    pltpu.sync_copy(srcp_hbm.at[wid], src_v)
    pltpu.sync_copy(dstp_hbm.at[wid], dst_v)

    def issue(j, rv, sg):
        pltpu.async_copy(aug_hbm.at[src_v.at[j]], rv, sg)

    def wait_gather(j, rv, sg):
        pltpu.make_async_copy(aug_hbm.at[src_v.at[j]], rv, sg).wait()

    issue(0, rows_v, sem)

    def chunk2(jj, carry):
        j0 = 2 * jj
        j1 = j0 + 1

        @pl.when(j1 < NCHK)
        def _():
            issue(j1, rows_v1, sem1)
        wait_gather(j0, rows_v, sem)
        pltpu.sync_copy(rows_v, acc_sh.at[dst_v.at[j0]], add=True)

        @pl.when(j1 < NCHK)
        def _():
            @pl.when(j1 + 1 < NCHK)
            def _():
                issue(j1 + 1, rows_v, sem)
            wait_gather(j1, rows_v1, sem1)
            pltpu.sync_copy(rows_v1, acc_sh.at[dst_v.at[j1]], add=True)
        return carry

    lax.fori_loop(0, (NCHK + 1) // 2, chunk2, 0)
    plsc.subcore_barrier()
    pltpu.sync_copy(acc_sh.at[pl.ds(s * RPS, RPS)],
                    acc_hbm.at[c, pl.ds(s * RPS, RPS)])


def _mid_body(aug_ref, acc_ref, gwa, gwb, gb, gg, gbe,
              nwp, nbp, ngp, nbep, w1a, w1bp, w1d, w1ep, b1,
              n_ref, a_ref, b_ref):
    hp = aug_ref[:, :128]
    ssum = acc_ref[0] + acc_ref[1]
    ah = ssum[:, :128]
    deg = ssum[:, 128:129]
    nrm = jnp.where(deg > 0, 1.0 / deg, 0.0)
    pre = hp @ gwa[...] + (ah * nrm) @ gwb[...] + gb[...]
    hg = jnp.maximum(_ln(pre, gg[...], gbe[...]), 0.0)
    nh = hg @ nwp[...] + nbp[...]
    cmask = (lax.broadcasted_iota(jnp.int32, (1, 128), 1) < 5).astype(jnp.float32)
    nm = jnp.sum(nh * cmask, axis=1, keepdims=True) * (1.0 / 5.0)
    d = (nh - nm) * cmask
    nv = jnp.sum(d * d, axis=1, keepdims=True) * (1.0 / 5.0)
    nf = d * lax.rsqrt(nv + EPS) * ngp[...] + nbep[...]
    n_ref[...] = nf
    a_ref[...] = hg @ w1a[...] + nf @ w1bp[...] + b1[...]
    b_ref[...] = hg @ w1d[...] + nf @ w1ep[...]


def _sc_gather_body(a_hbm, b_hbm, srcp_hbm, dstp_hbm, out_hbm,
                    src_v, dst_v, bufa0, bufb0, bufa1, bufb1,
                    sema0, semb0, sema1, semb1, semw0, semw1):
    c = lax.axis_index("c")
    s = lax.axis_index("s")
    wid = s * NC + c
    pltpu.sync_copy(srcp_hbm.at[wid], src_v)
    pltpu.sync_copy(dstp_hbm.at[wid], dst_v)
    base = wid * EPT

    def issue(j, ba, bb, sa, sb):
        pltpu.async_copy(a_hbm.at[src_v.at[j]], ba, sa)
        pltpu.async_copy(b_hbm.at[dst_v.at[j]], bb, sb)

    def wait_gather(j, ba, bb, sa, sb):
        pltpu.make_async_copy(a_hbm.at[src_v.at[j]], ba, sa).wait()
        pltpu.make_async_copy(b_hbm.at[dst_v.at[j]], bb, sb).wait()

    def add_rows(ba, bb):
        def row(r2, cc):
            for u in range(2):
                r = r2 * 2 + u
                for k in range(DH // 16):
                    sl = pl.ds(k * 16, 16)
                    ba[r, sl] = ba[r, sl] + bb[r, sl]
            return cc
        lax.fori_loop(0, CH // 2, row, 0)

    def drain_wb(j, ba, sw):
        pltpu.make_async_copy(ba, out_hbm.at[pl.ds(base + j * CH, CH)], sw).wait()

    issue(0, bufa0, bufb0, sema0, semb0)

    def outer(jj, carry):
        j0 = 2 * jj
        j1 = j0 + 1

        @pl.when(j1 < NCHK)
        def _():
            issue(j1, bufa1, bufb1, sema1, semb1)
        wait_gather(j0, bufa0, bufb0, sema0, semb0)
        add_rows(bufa0, bufb0)
        pltpu.async_copy(bufa0, out_hbm.at[pl.ds(base + j0 * CH, CH)], semw0)

        @pl.when(j1 < NCHK)
        def _():
            @pl.when(j1 + 1 < NCHK)
            def _():
                drain_wb(j0, bufa0, semw0)
                issue(j1 + 1, bufa0, bufb0, sema0, semb0)
            wait_gather(j1, bufa1, bufb1, sema1, semb1)
            add_rows(bufa1, bufb1)
            pltpu.sync_copy(bufa1, out_hbm.at[pl.ds(base + j1 * CH, CH)])

        @pl.when(j1 >= NCHK)
        def _():
            drain_wb(j0, bufa0, semw0)
        return carry

    lax.fori_loop(0, (NCHK + 1) // 2, outer, 0)


def _edge_body(t_ref, ea_ref, w1cp, lg, lb, w2p, b2p, out_ref):
    x = t_ref[...] + ea_ref[...] @ w1cp[...]
    y = jnp.maximum(_ln(x, lg[...], lb[...]), 0.0)
    e = y @ w2p[...] + b2p[...]
    out_ref[...] = e[:, :2]


def _full(shape):
    return pl.BlockSpec(shape, lambda i: tuple(0 for _ in shape))


def kernel(h, edge_index, edge_attr, params):
    f32 = jnp.float32
    hpad = jnp.zeros((NP, 128), f32).at[:NN].set(h)
    srcp = edge_index[0].reshape(NW, NCHK, CH)
    dstp = edge_index[1].reshape(NW, NCHK, CH)
    zr = jnp.zeros((RPS, RW), f32)

    p0, p1 = params["proj"][0], params["proj"][1]
    row = lambda v: v.reshape(1, -1)

    aug = pl.pallas_call(
        _proj_body,
        grid=(NBLK,),
        in_specs=[pl.BlockSpec((BLK, 128), lambda i: (i, 0)),
                  _full((64, 64)), _full((1, 64)), _full((1, 64)), _full((1, 64)),
                  _full((64, 64)), _full((1, 64)), _full((1, 64)), _full((1, 64))],
        out_specs=pl.BlockSpec((BLK, RW), lambda i: (i, 0)),
        out_shape=jax.ShapeDtypeStruct((NP, RW), f32),
    )(hpad, p0["W"], row(p0["b"]), row(p0["g"]), row(p0["beta"]),
      p1["W"], row(p1["b"]), row(p1["g"]), row(p1["beta"]))

    mesh = plsc.VectorSubcoreMesh(core_axis_name="c", subcore_axis_name="s",
                                  num_cores=NC, num_subcores=NS)
    scatter = pl.kernel(
        _sc_scatter_body,
        out_type=jax.ShapeDtypeStruct((NC, NP, RW), f32),
        mesh=mesh,
        compiler_params=pltpu.CompilerParams(use_tc_tiling_on_sc=False),
        scratch_types=[
            pltpu.VMEM_SHARED((NP, RW), f32),
            pltpu.VMEM((NCHK, CH), jnp.int32),
            pltpu.VMEM((NCHK, CH), jnp.int32),
            pltpu.VMEM((CH, RW), f32),
            pltpu.VMEM((CH, RW), f32),
            pltpu.SemaphoreType.DMA,
            pltpu.SemaphoreType.DMA,
        ],
    )
    acc = scatter(aug, srcp, dstp, zr)

    W1 = params["W1"]
    nWp = jnp.zeros((128, 128), f32).at[:, :5].set(params["node_W"])
    nbp = jnp.zeros((1, 128), f32).at[0, :5].set(params["node_b"])
    ngp = jnp.zeros((1, 128), f32).at[0, :5].set(params["node_g"])
    nbep = jnp.zeros((1, 128), f32).at[0, :5].set(params["node_beta"])
    W1bp = jnp.zeros((128, DH), f32).at[:5].set(W1[128:133])
    W1ep = jnp.zeros((128, DH), f32).at[:5].set(W1[265:270])
    nfull, tabA, tabB = pl.pallas_call(
        _mid_body,
        grid=(NBLK,),
        in_specs=[pl.BlockSpec((BLK, RW), lambda i: (i, 0)),
                  pl.BlockSpec((NC, BLK, RW), lambda i: (0, i, 0)),
                  _full((128, 128)), _full((128, 128)), _full((1, 128)),
                  _full((1, 128)), _full((1, 128)),
                  _full((128, 128)), _full((1, 128)), _full((1, 128)), _full((1, 128)),
                  _full((128, DH)), _full((128, DH)), _full((128, DH)),
                  _full((128, DH)), _full((1, DH))],
        out_specs=[pl.BlockSpec((BLK, 128), lambda i: (i, 0)),
                   pl.BlockSpec((BLK, DH), lambda i: (i, 0)),
                   pl.BlockSpec((BLK, DH), lambda i: (i, 0))],
        out_shape=[jax.ShapeDtypeStruct((NP, 128), f32),
                   jax.ShapeDtypeStruct((NP, DH), f32),
                   jax.ShapeDtypeStruct((NP, DH), f32)],
    )(aug, acc,
      params["gcn_W"][:128], params["gcn_W"][128:], row(params["gcn_b"]),
      row(params["gcn_g"]), row(params["gcn_beta"]),
      nWp, nbp, ngp, nbep,
      W1[0:128], W1bp, W1[137:265], W1ep, row(params["b1"]))

    gather = pl.kernel(
        _sc_gather_body,
        out_type=jax.ShapeDtypeStruct((EE, DH), f32),
        mesh=mesh,
        scratch_types=[
            pltpu.VMEM((NCHK, CH), jnp.int32),
            pltpu.VMEM((NCHK, CH), jnp.int32),
            pltpu.VMEM((CH, DH), f32),
            pltpu.VMEM((CH, DH), f32),
            pltpu.VMEM((CH, DH), f32),
            pltpu.VMEM((CH, DH), f32),
            pltpu.SemaphoreType.DMA,
            pltpu.SemaphoreType.DMA,
            pltpu.SemaphoreType.DMA,
            pltpu.SemaphoreType.DMA,
            pltpu.SemaphoreType.DMA,
            pltpu.SemaphoreType.DMA,
        ],
    )
    tsum = gather(tabA, tabB, srcp, dstp)

    W2p = jnp.zeros((DH, 128), f32).at[:, :2].set(params["W2"])
    b2p = jnp.zeros((1, 128), f32).at[0, :2].set(params["b2"])
    e = pl.pallas_call(
        _edge_body,
        grid=(NEB,),
        in_specs=[pl.BlockSpec((EBLK, DH), lambda i: (i, 0)),
                  pl.BlockSpec((EBLK, 4), lambda i: (i, 0)),
                  _full((4, DH)), _full((1, DH)), _full((1, DH)),
                  _full((DH, 128)), _full((1, 128))],
        out_specs=pl.BlockSpec((EBLK, 2), lambda i: (i, 0)),
        out_shape=jax.ShapeDtypeStruct((EE, 2), f32),
    )(tsum, edge_attr, W1[133:137],
      row(params["ln1_g"]), row(params["ln1_b"]), W2p, b2p)

    n = nfull[:NN, :5]
    return (n, e)


# speedup vs baseline: 6.2760x; 1.1241x over previous
import functools

import jax
import jax.numpy as jnp
from jax import lax
from jax.experimental import pallas as pl
from jax.experimental.pallas import tpu as pltpu
from jax.experimental.pallas import tpu_sc as plsc

NN = 10000
NP = 10240
RW = 136
EE = 320000
NC, NS = 2, 16
NW = NC * NS
EPT = EE // NW
CH = 80
NCHK = EPT // CH
RPS = NP // NS
DH = 256
NBLK = 8
BLK = NP // NBLK
EBLK = 3200
NEB = EE // EBLK
EPS = 1e-5


def _ln(x, g, b):
    m = jnp.mean(x, axis=1, keepdims=True)
    v = jnp.mean((x - m) ** 2, axis=1, keepdims=True)
    return (x - m) * lax.rsqrt(v + EPS) * g + b


def _proj_body(h_ref, w0, b0, g0, be0, w1, b1, g1, be1, out_ref):
    h = h_ref[...]
    z0 = jnp.maximum(_ln(h[:, :64] @ w0[...] + b0[...], g0[...], be0[...]), 0.0)
    z1 = jnp.maximum(_ln(h[:, 64:] @ w1[...] + b1[...], g1[...], be1[...]), 0.0)
    out_ref[:, 0:64] = z0
    out_ref[:, 64:128] = z1
    col = lax.broadcasted_iota(jnp.int32, (BLK, 8), 1)
    out_ref[:, 128:136] = jnp.where(col == 0, 1.0, 0.0)


def _sc_scatter_body(aug_hbm, srcp_hbm, dstp_hbm, zr_hbm, acc_hbm,
                     acc_sh, src_v, dst_v, rows_v, rows_v1, sem, sem1):
    c = lax.axis_index("c")
    s = lax.axis_index("s")
    wid = s * NC + c
    pltpu.sync_copy(zr_hbm, acc_sh.at[pl.ds(s * RPS, RPS)])
    plsc.subcore_barrier()
    pltpu.sync_copy(srcp_hbm.at[wid], src_v)
    pltpu.sync_copy(dstp_hbm.at[wid], dst_v)

    def issue(j, rv, sg):
        pltpu.async_copy(aug_hbm.at[src_v.at[j]], rv, sg)

    def wait_gather(j, rv, sg):
        pltpu.make_async_copy(aug_hbm.at[src_v.at[j]], rv, sg).wait()

    issue(0, rows_v, sem)

    def chunk2(jj, carry):
        j0 = 2 * jj
        j1 = j0 + 1

        @pl.when(j1 < NCHK)
        def _():
            issue(j1, rows_v1, sem1)
        wait_gather(j0, rows_v, sem)
        pltpu.sync_copy(rows_v, acc_sh.at[dst_v.at[j0]], add=True)

        @pl.when(j1 < NCHK)
        def _():
            @pl.when(j1 + 1 < NCHK)
            def _():
                issue(j1 + 1, rows_v, sem)
            wait_gather(j1, rows_v1, sem1)
            pltpu.sync_copy(rows_v1, acc_sh.at[dst_v.at[j1]], add=True)
        return carry

    lax.fori_loop(0, (NCHK + 1) // 2, chunk2, 0)
    plsc.subcore_barrier()
    pltpu.sync_copy(acc_sh.at[pl.ds(s * RPS, RPS)],
                    acc_hbm.at[c, pl.ds(s * RPS, RPS)])


def _mid_body(aug_ref, acc_ref, gwa, gwb, gb, gg, gbe,
              nwp, nbp, ngp, nbep, w1a, w1bp, w1d, w1ep, b1,
              n_ref, a_ref, b_ref):
    hp = aug_ref[:, :128]
    ssum = acc_ref[0] + acc_ref[1]
    ah = ssum[:, :128]
    deg = ssum[:, 128:129]
    nrm = jnp.where(deg > 0, 1.0 / deg, 0.0)
    pre = hp @ gwa[...] + (ah * nrm) @ gwb[...] + gb[...]
    hg = jnp.maximum(_ln(pre, gg[...], gbe[...]), 0.0)
    nh = hg @ nwp[...] + nbp[...]
    cmask = (lax.broadcasted_iota(jnp.int32, (1, 128), 1) < 5).astype(jnp.float32)
    nm = jnp.sum(nh * cmask, axis=1, keepdims=True) * (1.0 / 5.0)
    d = (nh - nm) * cmask
    nv = jnp.sum(d * d, axis=1, keepdims=True) * (1.0 / 5.0)
    nf = d * lax.rsqrt(nv + EPS) * ngp[...] + nbep[...]
    n_ref[...] = nf
    a_ref[...] = hg @ w1a[...] + nf @ w1bp[...] + b1[...]
    b_ref[...] = hg @ w1d[...] + nf @ w1ep[...]


def _sc_gather_body(a_hbm, b_hbm, srcp_hbm, dstp_hbm, out_hbm,
                    src_v, dst_v, bufa0, bufb0, bufa1, bufb1,
                    sema0, semb0, sema1, semb1, semw0, semw1):
    c = lax.axis_index("c")
    s = lax.axis_index("s")
    wid = s * NC + c
    pltpu.sync_copy(srcp_hbm.at[wid], src_v)
    pltpu.sync_copy(dstp_hbm.at[wid], dst_v)
    base = wid * EPT

    def issue(j, ba, bb, sa, sb):
        pltpu.async_copy(a_hbm.at[src_v.at[j]], ba, sa)
        pltpu.async_copy(b_hbm.at[dst_v.at[j]], bb, sb)

    def wait_gather(j, ba, bb, sa, sb):
        pltpu.make_async_copy(a_hbm.at[src_v.at[j]], ba, sa).wait()
        pltpu.make_async_copy(b_hbm.at[dst_v.at[j]], bb, sb).wait()

    def add_rows(ba, bb):
        def row(r2, cc):
            for u in range(2):
                r = r2 * 2 + u
                for k in range(DH // 16):
                    sl = pl.ds(k * 16, 16)
                    ba[r, sl] = ba[r, sl] + bb[r, sl]
            return cc
        lax.fori_loop(0, CH // 2, row, 0)

    def drain_wb(j, ba, sw):
        pltpu.make_async_copy(ba, out_hbm.at[pl.ds(base + j * CH, CH)], sw).wait()

    issue(0, bufa0, bufb0, sema0, semb0)

    def outer(jj, carry):
        j0 = 2 * jj
        j1 = j0 + 1

        @pl.when(j1 < NCHK)
        def _():
            issue(j1, bufa1, bufb1, sema1, semb1)
        wait_gather(j0, bufa0, bufb0, sema0, semb0)
        add_rows(bufa0, bufb0)
        pltpu.async_copy(bufa0, out_hbm.at[pl.ds(base + j0 * CH, CH)], semw0)

        @pl.when(j1 < NCHK)
        def _():
            @pl.when(j1 + 1 < NCHK)
            def _():
                drain_wb(j0, bufa0, semw0)
                issue(j1 + 1, bufa0, bufb0, sema0, semb0)
            wait_gather(j1, bufa1, bufb1, sema1, semb1)
            add_rows(bufa1, bufb1)
            pltpu.sync_copy(bufa1, out_hbm.at[pl.ds(base + j1 * CH, CH)])

        @pl.when(j1 >= NCHK)
        def _():
            drain_wb(j0, bufa0, semw0)
        return carry

    lax.fori_loop(0, (NCHK + 1) // 2, outer, 0)


def _edge_body(t_ref, eat_ref, w1c, lg, lb, w2t8, b2c, out_ref):
    x = t_ref[...] + lax.dot_general(eat_ref[...], w1c[...],
                                     (((0,), (0,)), ((), ())))
    y = jnp.maximum(_ln(x, lg[...], lb[...]), 0.0)
    e8 = lax.dot_general(w2t8[...], y, (((1,), (1,)), ((), ())))
    out_ref[...] = e8[:2, :] + b2c[...]


def _full(shape):
    return pl.BlockSpec(shape, lambda i: tuple(0 for _ in shape))


def kernel(h, edge_index, edge_attr, params):
    f32 = jnp.float32
    hpad = jnp.zeros((NP, 128), f32).at[:NN].set(h)
    srcp = edge_index[0].reshape(NW, NCHK, CH)
    dstp = edge_index[1].reshape(NW, NCHK, CH)
    zr = jnp.zeros((RPS, RW), f32)

    p0, p1 = params["proj"][0], params["proj"][1]
    row = lambda v: v.reshape(1, -1)

    aug = pl.pallas_call(
        _proj_body,
        grid=(NBLK,),
        in_specs=[pl.BlockSpec((BLK, 128), lambda i: (i, 0)),
                  _full((64, 64)), _full((1, 64)), _full((1, 64)), _full((1, 64)),
                  _full((64, 64)), _full((1, 64)), _full((1, 64)), _full((1, 64))],
        out_specs=pl.BlockSpec((BLK, RW), lambda i: (i, 0)),
        out_shape=jax.ShapeDtypeStruct((NP, RW), f32),
    )(hpad, p0["W"], row(p0["b"]), row(p0["g"]), row(p0["beta"]),
      p1["W"], row(p1["b"]), row(p1["g"]), row(p1["beta"]))

    mesh = plsc.VectorSubcoreMesh(core_axis_name="c", subcore_axis_name="s",
                                  num_cores=NC, num_subcores=NS)
    scatter = pl.kernel(
        _sc_scatter_body,
        out_type=jax.ShapeDtypeStruct((NC, NP, RW), f32),
        mesh=mesh,
        compiler_params=pltpu.CompilerParams(use_tc_tiling_on_sc=False),
        scratch_types=[
            pltpu.VMEM_SHARED((NP, RW), f32),
            pltpu.VMEM((NCHK, CH), jnp.int32),
            pltpu.VMEM((NCHK, CH), jnp.int32),
            pltpu.VMEM((CH, RW), f32),
            pltpu.VMEM((CH, RW), f32),
            pltpu.SemaphoreType.DMA,
            pltpu.SemaphoreType.DMA,
        ],
    )
    acc = scatter(aug, srcp, dstp, zr)

    W1 = params["W1"]
    nWp = jnp.zeros((128, 128), f32).at[:, :5].set(params["node_W"])
    nbp = jnp.zeros((1, 128), f32).at[0, :5].set(params["node_b"])
    ngp = jnp.zeros((1, 128), f32).at[0, :5].set(params["node_g"])
    nbep = jnp.zeros((1, 128), f32).at[0, :5].set(params["node_beta"])
    W1bp = jnp.zeros((128, DH), f32).at[:5].set(W1[128:133])
    W1ep = jnp.zeros((128, DH), f32).at[:5].set(W1[265:270])
    nfull, tabA, tabB = pl.pallas_call(
        _mid_body,
        grid=(NBLK,),
        in_specs=[pl.BlockSpec((BLK, RW), lambda i: (i, 0)),
                  pl.BlockSpec((NC, BLK, RW), lambda i: (0, i, 0)),
                  _full((128, 128)), _full((128, 128)), _full((1, 128)),
                  _full((1, 128)), _full((1, 128)),
                  _full((128, 128)), _full((1, 128)), _full((1, 128)), _full((1, 128)),
                  _full((128, DH)), _full((128, DH)), _full((128, DH)),
                  _full((128, DH)), _full((1, DH))],
        out_specs=[pl.BlockSpec((BLK, 128), lambda i: (i, 0)),
                   pl.BlockSpec((BLK, DH), lambda i: (i, 0)),
                   pl.BlockSpec((BLK, DH), lambda i: (i, 0))],
        out_shape=[jax.ShapeDtypeStruct((NP, 128), f32),
                   jax.ShapeDtypeStruct((NP, DH), f32),
                   jax.ShapeDtypeStruct((NP, DH), f32)],
    )(aug, acc,
      params["gcn_W"][:128], params["gcn_W"][128:], row(params["gcn_b"]),
      row(params["gcn_g"]), row(params["gcn_beta"]),
      nWp, nbp, ngp, nbep,
      W1[0:128], W1bp, W1[137:265], W1ep, row(params["b1"]))

    gather = pl.kernel(
        _sc_gather_body,
        out_type=jax.ShapeDtypeStruct((EE, DH), f32),
        mesh=mesh,
        scratch_types=[
            pltpu.VMEM((NCHK, CH), jnp.int32),
            pltpu.VMEM((NCHK, CH), jnp.int32),
            pltpu.VMEM((CH, DH), f32),
            pltpu.VMEM((CH, DH), f32),
            pltpu.VMEM((CH, DH), f32),
            pltpu.VMEM((CH, DH), f32),
            pltpu.SemaphoreType.DMA,
            pltpu.SemaphoreType.DMA,
            pltpu.SemaphoreType.DMA,
            pltpu.SemaphoreType.DMA,
            pltpu.SemaphoreType.DMA,
            pltpu.SemaphoreType.DMA,
        ],
    )
    tsum = gather(tabA, tabB, srcp, dstp)

    w2t8 = jnp.zeros((8, DH), f32).at[:2].set(params["W2"].T)
    b2c = params["b2"].reshape(2, 1)
    eT = pl.pallas_call(
        _edge_body,
        grid=(NEB,),
        in_specs=[pl.BlockSpec((EBLK, DH), lambda i: (i, 0)),
                  pl.BlockSpec((4, EBLK), lambda i: (0, i)),
                  _full((4, DH)), _full((1, DH)), _full((1, DH)),
                  _full((8, DH)), _full((2, 1))],
        out_specs=pl.BlockSpec((2, EBLK), lambda i: (0, i)),
        out_shape=jax.ShapeDtypeStruct((2, EE), f32),
    )(tsum, edge_attr.T, W1[133:137],
      row(params["ln1_g"]), row(params["ln1_b"]), w2t8, b2c)

    n = nfull[:NN, :5]
    return (n, eT.T)


# speedup vs baseline: 6.9602x; 1.1090x over previous
import functools

import jax
import jax.numpy as jnp
from jax import lax
from jax.experimental import pallas as pl
from jax.experimental.pallas import tpu as pltpu
from jax.experimental.pallas import tpu_sc as plsc

NN = 10000
NP = 10240
RW = 136
EE = 320000
NC, NS = 2, 16
NW = NC * NS
EPT = EE // NW
CH = 80
NCHK = EPT // CH
RPS = NP // NS
DH = 256
NBLK = 8
BLK = NP // NBLK
EBLK = 3200
EPS = 1e-5
NSLC = 5
SEE = EE // NSLC
EPTS = SEE // NW
NCHK2 = EPTS // CH
NEB2 = SEE // EBLK


def _ln(x, g, b):
    m = jnp.mean(x, axis=1, keepdims=True)
    v = jnp.mean((x - m) ** 2, axis=1, keepdims=True)
    return (x - m) * lax.rsqrt(v + EPS) * g + b


def _proj_body(h_ref, w0, b0, g0, be0, w1, b1, g1, be1, out_ref):
    h = h_ref[...]
    z0 = jnp.maximum(_ln(h[:, :64] @ w0[...] + b0[...], g0[...], be0[...]), 0.0)
    z1 = jnp.maximum(_ln(h[:, 64:] @ w1[...] + b1[...], g1[...], be1[...]), 0.0)
    out_ref[:, 0:64] = z0
    out_ref[:, 64:128] = z1
    col = lax.broadcasted_iota(jnp.int32, (BLK, 8), 1)
    out_ref[:, 128:136] = jnp.where(col == 0, 1.0, 0.0)


def _sc_scatter_body(aug_hbm, srcp_hbm, dstp_hbm, zr_hbm, acc_hbm,
                     acc_sh, src_v, dst_v, rows_v, rows_v1, sem, sem1):
    c = lax.axis_index("c")
    s = lax.axis_index("s")
    wid = s * NC + c
    pltpu.sync_copy(zr_hbm, acc_sh.at[pl.ds(s * RPS, RPS)])
    plsc.subcore_barrier()
    pltpu.sync_copy(srcp_hbm.at[wid], src_v)
    pltpu.sync_copy(dstp_hbm.at[wid], dst_v)

    def issue(j, rv, sg):
        pltpu.async_copy(aug_hbm.at[src_v.at[j]], rv, sg)

    def wait_gather(j, rv, sg):
        pltpu.make_async_copy(aug_hbm.at[src_v.at[j]], rv, sg).wait()

    issue(0, rows_v, sem)

    def chunk2(jj, carry):
        j0 = 2 * jj
        j1 = j0 + 1

        @pl.when(j1 < NCHK)
        def _():
            issue(j1, rows_v1, sem1)
        wait_gather(j0, rows_v, sem)
        pltpu.sync_copy(rows_v, acc_sh.at[dst_v.at[j0]], add=True)

        @pl.when(j1 < NCHK)
        def _():
            @pl.when(j1 + 1 < NCHK)
            def _():
                issue(j1 + 1, rows_v, sem)
            wait_gather(j1, rows_v1, sem1)
            pltpu.sync_copy(rows_v1, acc_sh.at[dst_v.at[j1]], add=True)
        return carry

    lax.fori_loop(0, (NCHK + 1) // 2, chunk2, 0)
    plsc.subcore_barrier()
    pltpu.sync_copy(acc_sh.at[pl.ds(s * RPS, RPS)],
                    acc_hbm.at[c, pl.ds(s * RPS, RPS)])


def _mid_body(aug_ref, acc_ref, gwa, gwb, gb, gg, gbe,
              nwp, nbp, ngp, nbep, w1a, w1bp, w1d, w1ep, b1,
              n_ref, a_ref, b_ref):
    hp = aug_ref[:, :128]
    ssum = acc_ref[0] + acc_ref[1]
    ah = ssum[:, :128]
    deg = ssum[:, 128:129]
    nrm = jnp.where(deg > 0, 1.0 / deg, 0.0)
    pre = hp @ gwa[...] + (ah * nrm) @ gwb[...] + gb[...]
    hg = jnp.maximum(_ln(pre, gg[...], gbe[...]), 0.0)
    nh = hg @ nwp[...] + nbp[...]
    cmask = (lax.broadcasted_iota(jnp.int32, (1, 128), 1) < 5).astype(jnp.float32)
    nm = jnp.sum(nh * cmask, axis=1, keepdims=True) * (1.0 / 5.0)
    d = (nh - nm) * cmask
    nv = jnp.sum(d * d, axis=1, keepdims=True) * (1.0 / 5.0)
    nf = d * lax.rsqrt(nv + EPS) * ngp[...] + nbep[...]
    n_ref[...] = nf
    a_ref[...] = hg @ w1a[...] + nf @ w1bp[...] + b1[...]
    b_ref[...] = hg @ w1d[...] + nf @ w1ep[...]


def _sc_gather_body(a_hbm, b_hbm, srcp_hbm, dstp_hbm, out_hbm,
                    src_v, dst_v, bufa0, bufb0, bufa1, bufb1,
                    sema0, semb0, sema1, semb1, semw0, semw1):
    c = lax.axis_index("c")
    s = lax.axis_index("s")
    wid = s * NC + c
    pltpu.sync_copy(srcp_hbm.at[wid], src_v)
    pltpu.sync_copy(dstp_hbm.at[wid], dst_v)
    base = wid * EPTS

    def issue(j, ba, bb, sa, sb):
        pltpu.async_copy(a_hbm.at[src_v.at[j]], ba, sa)
        pltpu.async_copy(b_hbm.at[dst_v.at[j]], bb, sb)

    def wait_gather(j, ba, bb, sa, sb):
        pltpu.make_async_copy(a_hbm.at[src_v.at[j]], ba, sa).wait()
        pltpu.make_async_copy(b_hbm.at[dst_v.at[j]], bb, sb).wait()

    def add_rows(ba, bb):
        def row(r2, cc):
            for u in range(2):
                r = r2 * 2 + u
                for k in range(DH // 16):
                    sl = pl.ds(k * 16, 16)
                    ba[r, sl] = ba[r, sl] + bb[r, sl]
            return cc
        lax.fori_loop(0, CH // 2, row, 0)

    def drain_wb(j, ba, sw):
        pltpu.make_async_copy(ba, out_hbm.at[pl.ds(base + j * CH, CH)], sw).wait()

    issue(0, bufa0, bufb0, sema0, semb0)

    def outer(jj, carry):
        j0 = 2 * jj
        j1 = j0 + 1

        @pl.when(j1 < NCHK2)
        def _():
            issue(j1, bufa1, bufb1, sema1, semb1)
        wait_gather(j0, bufa0, bufb0, sema0, semb0)
        add_rows(bufa0, bufb0)
        pltpu.async_copy(bufa0, out_hbm.at[pl.ds(base + j0 * CH, CH)], semw0)

        @pl.when(j1 < NCHK2)
        def _():
            @pl.when(j1 + 1 < NCHK2)
            def _():
                drain_wb(j0, bufa0, semw0)
                issue(j1 + 1, bufa0, bufb0, sema0, semb0)
            wait_gather(j1, bufa1, bufb1, sema1, semb1)
            add_rows(bufa1, bufb1)
            pltpu.sync_copy(bufa1, out_hbm.at[pl.ds(base + j1 * CH, CH)])

        @pl.when(j1 >= NCHK2)
        def _():
            drain_wb(j0, bufa0, semw0)
        return carry

    lax.fori_loop(0, (NCHK2 + 1) // 2, outer, 0)


def _edge_body(t_ref, eat_ref, w1c, lg, lb, w2t8, b2c, out_ref):
    x = t_ref[...] + lax.dot_general(eat_ref[...], w1c[...],
                                     (((0,), (0,)), ((), ())))
    y = jnp.maximum(_ln(x, lg[...], lb[...]), 0.0)
    e8 = lax.dot_general(w2t8[...], y, (((1,), (1,)), ((), ())))
    out_ref[...] = e8[:2, :] + b2c[...]


def _full(shape):
    return pl.BlockSpec(shape, lambda i: tuple(0 for _ in shape))


def kernel(h, edge_index, edge_attr, params):
    f32 = jnp.float32
    hpad = jnp.zeros((NP, 128), f32).at[:NN].set(h)
    srcp = edge_index[0].reshape(NW, NCHK, CH)
    dstp = edge_index[1].reshape(NW, NCHK, CH)
    zr = jnp.zeros((RPS, RW), f32)

    p0, p1 = params["proj"][0], params["proj"][1]
    row = lambda v: v.reshape(1, -1)

    aug = pl.pallas_call(
        _proj_body,
        grid=(NBLK,),
        in_specs=[pl.BlockSpec((BLK, 128), lambda i: (i, 0)),
                  _full((64, 64)), _full((1, 64)), _full((1, 64)), _full((1, 64)),
                  _full((64, 64)), _full((1, 64)), _full((1, 64)), _full((1, 64))],
        out_specs=pl.BlockSpec((BLK, RW), lambda i: (i, 0)),
        out_shape=jax.ShapeDtypeStruct((NP, RW), f32),
    )(hpad, p0["W"], row(p0["b"]), row(p0["g"]), row(p0["beta"]),
      p1["W"], row(p1["b"]), row(p1["g"]), row(p1["beta"]))

    mesh = plsc.VectorSubcoreMesh(core_axis_name="c", subcore_axis_name="s",
                                  num_cores=NC, num_subcores=NS)
    scatter = pl.kernel(
        _sc_scatter_body,
        out_type=jax.ShapeDtypeStruct((NC, NP, RW), f32),
        mesh=mesh,
        compiler_params=pltpu.CompilerParams(use_tc_tiling_on_sc=False),
        scratch_types=[
            pltpu.VMEM_SHARED((NP, RW), f32),
            pltpu.VMEM((NCHK, CH), jnp.int32),
            pltpu.VMEM((NCHK, CH), jnp.int32),
            pltpu.VMEM((CH, RW), f32),
            pltpu.VMEM((CH, RW), f32),
            pltpu.SemaphoreType.DMA,
            pltpu.SemaphoreType.DMA,
        ],
    )
    acc = scatter(aug, srcp, dstp, zr)

    W1 = params["W1"]
    nWp = jnp.zeros((128, 128), f32).at[:, :5].set(params["node_W"])
    nbp = jnp.zeros((1, 128), f32).at[0, :5].set(params["node_b"])
    ngp = jnp.zeros((1, 128), f32).at[0, :5].set(params["node_g"])
    nbep = jnp.zeros((1, 128), f32).at[0, :5].set(params["node_beta"])
    W1bp = jnp.zeros((128, DH), f32).at[:5].set(W1[128:133])
    W1ep = jnp.zeros((128, DH), f32).at[:5].set(W1[265:270])
    nfull, tabA, tabB = pl.pallas_call(
        _mid_body,
        grid=(NBLK,),
        in_specs=[pl.BlockSpec((BLK, RW), lambda i: (i, 0)),
                  pl.BlockSpec((NC, BLK, RW), lambda i: (0, i, 0)),
                  _full((128, 128)), _full((128, 128)), _full((1, 128)),
                  _full((1, 128)), _full((1, 128)),
                  _full((128, 128)), _full((1, 128)), _full((1, 128)), _full((1, 128)),
                  _full((128, DH)), _full((128, DH)), _full((128, DH)),
                  _full((128, DH)), _full((1, DH))],
        out_specs=[pl.BlockSpec((BLK, 128), lambda i: (i, 0)),
                   pl.BlockSpec((BLK, DH), lambda i: (i, 0)),
                   pl.BlockSpec((BLK, DH), lambda i: (i, 0))],
        out_shape=[jax.ShapeDtypeStruct((NP, 128), f32),
                   jax.ShapeDtypeStruct((NP, DH), f32),
                   jax.ShapeDtypeStruct((NP, DH), f32)],
    )(aug, acc,
      params["gcn_W"][:128], params["gcn_W"][128:], row(params["gcn_b"]),
      row(params["gcn_g"]), row(params["gcn_beta"]),
      nWp, nbp, ngp, nbep,
      W1[0:128], W1bp, W1[137:265], W1ep, row(params["b1"]))

    gather = pl.kernel(
        _sc_gather_body,
        out_type=jax.ShapeDtypeStruct((SEE, DH), f32),
        mesh=mesh,
        scratch_types=[
            pltpu.VMEM((NCHK2, CH), jnp.int32),
            pltpu.VMEM((NCHK2, CH), jnp.int32),
            pltpu.VMEM((CH, DH), f32),
            pltpu.VMEM((CH, DH), f32),
            pltpu.VMEM((CH, DH), f32),
            pltpu.VMEM((CH, DH), f32),
            pltpu.SemaphoreType.DMA,
            pltpu.SemaphoreType.DMA,
            pltpu.SemaphoreType.DMA,
            pltpu.SemaphoreType.DMA,
            pltpu.SemaphoreType.DMA,
            pltpu.SemaphoreType.DMA,
        ],
    )
    srcp5 = edge_index[0].reshape(NSLC, NW, NCHK2, CH)
    dstp5 = edge_index[1].reshape(NSLC, NW, NCHK2, CH)
    eaT = edge_attr.T
    w2t8 = jnp.zeros((8, DH), f32).at[:2].set(params["W2"].T)
    b2c = params["b2"].reshape(2, 1)
    edge_tail = pl.pallas_call(
        _edge_body,
        grid=(NEB2,),
        in_specs=[pl.BlockSpec((EBLK, DH), lambda i: (i, 0)),
                  pl.BlockSpec((4, EBLK), lambda i: (0, i)),
                  _full((4, DH)), _full((1, DH)), _full((1, DH)),
                  _full((8, DH)), _full((2, 1))],
        out_specs=pl.BlockSpec((2, EBLK), lambda i: (0, i)),
        out_shape=jax.ShapeDtypeStruct((2, SEE), f32),
    )
    eTs = []
    for sl in range(NSLC):
        tsum = gather(tabA, tabB, srcp5[sl], dstp5[sl])
        eTs.append(edge_tail(
            tsum, lax.dynamic_slice_in_dim(eaT, sl * SEE, SEE, axis=1),
            W1[133:137], row(params["ln1_g"]), row(params["ln1_b"]),
            w2t8, b2c))
    eT = jnp.concatenate(eTs, axis=1)

    n = nfull[:NN, :5]
    return (n, eT.T)


# speedup vs baseline: 7.6603x; 1.1006x over previous
import functools

import jax
import jax.numpy as jnp
from jax import lax
from jax.experimental import pallas as pl
from jax.experimental.pallas import tpu as pltpu
from jax.experimental.pallas import tpu_sc as plsc

NN = 10000
NP = 10240
RW = 136
EE = 320000
NC, NS = 2, 16
NW = NC * NS
EPT = EE // NW
CH = 80
NCHK = EPT // CH
RPS = NP // NS
DH = 256
NBLK = 8
BLK = NP // NBLK
EBLK = 3200
EPS = 1e-5
NSLC = 5
SEE = EE // NSLC
EPTS = SEE // NW
NCHK2 = EPTS // CH
NEB2 = SEE // EBLK


def _ln(x, g, b):
    m = jnp.mean(x, axis=1, keepdims=True)
    v = jnp.mean((x - m) ** 2, axis=1, keepdims=True)
    return (x - m) * lax.rsqrt(v + EPS) * g + b


def _proj_body(h_ref, w0, b0, g0, be0, w1, b1, g1, be1, out_ref):
    h = h_ref[...]
    z0 = jnp.maximum(_ln(h[:, :64] @ w0[...] + b0[...], g0[...], be0[...]), 0.0)
    z1 = jnp.maximum(_ln(h[:, 64:] @ w1[...] + b1[...], g1[...], be1[...]), 0.0)
    out_ref[:, 0:64] = z0
    out_ref[:, 64:128] = z1
    col = lax.broadcasted_iota(jnp.int32, (BLK, 8), 1)
    out_ref[:, 128:136] = jnp.where(col == 0, 1.0, 0.0)


def _sc_scatter_body(aug_hbm, srcp_hbm, dstp_hbm, zr_hbm, acc_hbm,
                     acc_sh, src_v, dst_v, rows_v, rows_v1, sem, sem1):
    c = lax.axis_index("c")
    s = lax.axis_index("s")
    wid = s * NC + c
    pltpu.sync_copy(zr_hbm, acc_sh.at[pl.ds(s * RPS, RPS)])
    plsc.subcore_barrier()
    pltpu.sync_copy(srcp_hbm.at[wid], src_v)
    pltpu.sync_copy(dstp_hbm.at[wid], dst_v)

    def issue(j, rv, sg):
        pltpu.async_copy(aug_hbm.at[src_v.at[j]], rv, sg)

    def wait_gather(j, rv, sg):
        pltpu.make_async_copy(aug_hbm.at[src_v.at[j]], rv, sg).wait()

    issue(0, rows_v, sem)

    def chunk2(jj, carry):
        j0 = 2 * jj
        j1 = j0 + 1

        @pl.when(j1 < NCHK)
        def _():
            issue(j1, rows_v1, sem1)
        wait_gather(j0, rows_v, sem)
        pltpu.sync_copy(rows_v, acc_sh.at[dst_v.at[j0]], add=True)

        @pl.when(j1 < NCHK)
        def _():
            @pl.when(j1 + 1 < NCHK)
            def _():
                issue(j1 + 1, rows_v, sem)
            wait_gather(j1, rows_v1, sem1)
            pltpu.sync_copy(rows_v1, acc_sh.at[dst_v.at[j1]], add=True)
        return carry

    lax.fori_loop(0, (NCHK + 1) // 2, chunk2, 0)
    plsc.subcore_barrier()
    pltpu.sync_copy(acc_sh.at[pl.ds(s * RPS, RPS)],
                    acc_hbm.at[c, pl.ds(s * RPS, RPS)])


def _mid_body(aug_ref, acc_ref, gwa, gwb, gb, gg, gbe,
              nwp, nbp, ngp, nbep, w1a, w1bp, w1d, w1ep, b1,
              n_ref, a_ref, b_ref):
    hp = aug_ref[:, :128]
    ssum = acc_ref[0] + acc_ref[1]
    ah = ssum[:, :128]
    deg = ssum[:, 128:129]
    nrm = jnp.where(deg > 0, 1.0 / deg, 0.0)
    pre = hp @ gwa[...] + (ah * nrm) @ gwb[...] + gb[...]
    hg = jnp.maximum(_ln(pre, gg[...], gbe[...]), 0.0)
    nh = hg @ nwp[...] + nbp[...]
    cmask = (lax.broadcasted_iota(jnp.int32, (1, 128), 1) < 5).astype(jnp.float32)
    nm = jnp.sum(nh * cmask, axis=1, keepdims=True) * (1.0 / 5.0)
    d = (nh - nm) * cmask
    nv = jnp.sum(d * d, axis=1, keepdims=True) * (1.0 / 5.0)
    nf = d * lax.rsqrt(nv + EPS) * ngp[...] + nbep[...]
    n_ref[...] = nf
    a_ref[...] = (hg @ w1a[...] + nf @ w1bp[...] + b1[...]).astype(jnp.bfloat16)
    b_ref[...] = (hg @ w1d[...] + nf @ w1ep[...]).astype(jnp.bfloat16)


def _sc_gather_body(a_hbm, b_hbm, srcp_hbm, dstp_hbm, out_hbm,
                    src_v, dst_v, bufa0, bufb0, bufa1, bufb1,
                    sema0, semb0, sema1, semb1, semw0, semw1):
    c = lax.axis_index("c")
    s = lax.axis_index("s")
    wid = s * NC + c
    pltpu.sync_copy(srcp_hbm.at[wid], src_v)
    pltpu.sync_copy(dstp_hbm.at[wid], dst_v)
    base = wid * EPTS

    def issue(j, ba, bb, sa, sb):
        pltpu.async_copy(a_hbm.at[src_v.at[j]], ba, sa)
        pltpu.async_copy(b_hbm.at[dst_v.at[j]], bb, sb)

    def wait_gather(j, ba, bb, sa, sb):
        pltpu.make_async_copy(a_hbm.at[src_v.at[j]], ba, sa).wait()
        pltpu.make_async_copy(b_hbm.at[dst_v.at[j]], bb, sb).wait()

    def add_rows(ba, bb):
        def row(r2, cc):
            for u in range(2):
                r = r2 * 2 + u
                for k in range(128 // 16):
                    sl = pl.ds(k * 16, 16)
                    s = (plsc.bitcast(ba[r, sl], jnp.bfloat16) +
                         plsc.bitcast(bb[r, sl], jnp.bfloat16))
                    ba[r, sl] = plsc.bitcast(s, jnp.float32)
            return cc
        lax.fori_loop(0, CH // 2, row, 0)

    def drain_wb(j, ba, sw):
        pltpu.make_async_copy(ba, out_hbm.at[pl.ds(base + j * CH, CH)], sw).wait()

    issue(0, bufa0, bufb0, sema0, semb0)

    def outer(jj, carry):
        j0 = 2 * jj
        j1 = j0 + 1

        @pl.when(j1 < NCHK2)
        def _():
            issue(j1, bufa1, bufb1, sema1, semb1)
        wait_gather(j0, bufa0, bufb0, sema0, semb0)
        add_rows(bufa0, bufb0)
        pltpu.async_copy(bufa0, out_hbm.at[pl.ds(base + j0 * CH, CH)], semw0)

        @pl.when(j1 < NCHK2)
        def _():
            @pl.when(j1 + 1 < NCHK2)
            def _():
                drain_wb(j0, bufa0, semw0)
                issue(j1 + 1, bufa0, bufb0, sema0, semb0)
            wait_gather(j1, bufa1, bufb1, sema1, semb1)
            add_rows(bufa1, bufb1)
            pltpu.sync_copy(bufa1, out_hbm.at[pl.ds(base + j1 * CH, CH)])

        @pl.when(j1 >= NCHK2)
        def _():
            drain_wb(j0, bufa0, semw0)
        return carry

    lax.fori_loop(0, (NCHK2 + 1) // 2, outer, 0)


def _edge_body(t_ref, eat_ref, w1ce, w1co, lg, lb, w2t8, b2c, out_ref):
    wi = lax.bitcast_convert_type(t_ref[...], jnp.int32)
    te = lax.bitcast_convert_type(wi << 16, jnp.float32)
    to = lax.bitcast_convert_type(wi & jnp.int32(-65536), jnp.float32)
    eat = eat_ref[...]
    x0 = te + lax.dot_general(eat, w1ce[...], (((0,), (0,)), ((), ())))
    x1 = to + lax.dot_general(eat, w1co[...], (((0,), (0,)), ((), ())))
    m = (jnp.sum(x0, axis=1, keepdims=True) +
         jnp.sum(x1, axis=1, keepdims=True)) * (1.0 / DH)
    d0 = x0 - m
    d1 = x1 - m
    v = (jnp.sum(d0 * d0, axis=1, keepdims=True) +
         jnp.sum(d1 * d1, axis=1, keepdims=True)) * (1.0 / DH)
    rs = lax.rsqrt(v + EPS)
    lgv = lg[...]
    lbv = lb[...]
    y0 = jnp.maximum(d0 * rs * lgv[:, :128] + lbv[:, :128], 0.0)
    y1 = jnp.maximum(d1 * rs * lgv[:, 128:] + lbv[:, 128:], 0.0)
    w2v = w2t8[...]
    e8 = (lax.dot_general(w2v[:, :128], y0, (((1,), (1,)), ((), ()))) +
          lax.dot_general(w2v[:, 128:], y1, (((1,), (1,)), ((), ()))))
    out_ref[...] = e8[:2, :] + b2c[...]


def _full(shape):
    return pl.BlockSpec(shape, lambda i: tuple(0 for _ in shape))


def kernel(h, edge_index, edge_attr, params):
    f32 = jnp.float32
    hpad = jnp.zeros((NP, 128), f32).at[:NN].set(h)
    srcp = edge_index[0].reshape(NW, NCHK, CH)
    dstp = edge_index[1].reshape(NW, NCHK, CH)
    zr = jnp.zeros((RPS, RW), f32)

    p0, p1 = params["proj"][0], params["proj"][1]
    row = lambda v: v.reshape(1, -1)

    aug = pl.pallas_call(
        _proj_body,
        grid=(NBLK,),
        in_specs=[pl.BlockSpec((BLK, 128), lambda i: (i, 0)),
                  _full((64, 64)), _full((1, 64)), _full((1, 64)), _full((1, 64)),
                  _full((64, 64)), _full((1, 64)), _full((1, 64)), _full((1, 64))],
        out_specs=pl.BlockSpec((BLK, RW), lambda i: (i, 0)),
        out_shape=jax.ShapeDtypeStruct((NP, RW), f32),
    )(hpad, p0["W"], row(p0["b"]), row(p0["g"]), row(p0["beta"]),
      p1["W"], row(p1["b"]), row(p1["g"]), row(p1["beta"]))

    mesh = plsc.VectorSubcoreMesh(core_axis_name="c", subcore_axis_name="s",
                                  num_cores=NC, num_subcores=NS)
    scatter = pl.kernel(
        _sc_scatter_body,
        out_type=jax.ShapeDtypeStruct((NC, NP, RW), f32),
        mesh=mesh,
        compiler_params=pltpu.CompilerParams(use_tc_tiling_on_sc=False),
        scratch_types=[
            pltpu.VMEM_SHARED((NP, RW), f32),
            pltpu.VMEM((NCHK, CH), jnp.int32),
            pltpu.VMEM((NCHK, CH), jnp.int32),
            pltpu.VMEM((CH, RW), f32),
            pltpu.VMEM((CH, RW), f32),
            pltpu.SemaphoreType.DMA,
            pltpu.SemaphoreType.DMA,
        ],
    )
    acc = scatter(aug, srcp, dstp, zr)

    W1 = params["W1"]
    nWp = jnp.zeros((128, 128), f32).at[:, :5].set(params["node_W"])
    nbp = jnp.zeros((1, 128), f32).at[0, :5].set(params["node_b"])
    ngp = jnp.zeros((1, 128), f32).at[0, :5].set(params["node_g"])
    nbep = jnp.zeros((1, 128), f32).at[0, :5].set(params["node_beta"])
    W1bp = jnp.zeros((128, DH), f32).at[:5].set(W1[128:133])
    W1ep = jnp.zeros((128, DH), f32).at[:5].set(W1[265:270])
    nfull, tabA, tabB = pl.pallas_call(
        _mid_body,
        grid=(NBLK,),
        in_specs=[pl.BlockSpec((BLK, RW), lambda i: (i, 0)),
                  pl.BlockSpec((NC, BLK, RW), lambda i: (0, i, 0)),
                  _full((128, 128)), _full((128, 128)), _full((1, 128)),
                  _full((1, 128)), _full((1, 128)),
                  _full((128, 128)), _full((1, 128)), _full((1, 128)), _full((1, 128)),
                  _full((128, DH)), _full((128, DH)), _full((128, DH)),
                  _full((128, DH)), _full((1, DH))],
        out_specs=[pl.BlockSpec((BLK, 128), lambda i: (i, 0)),
                   pl.BlockSpec((BLK, DH), lambda i: (i, 0)),
                   pl.BlockSpec((BLK, DH), lambda i: (i, 0))],
        out_shape=[jax.ShapeDtypeStruct((NP, 128), f32),
                   jax.ShapeDtypeStruct((NP, DH), jnp.bfloat16),
                   jax.ShapeDtypeStruct((NP, DH), jnp.bfloat16)],
    )(aug, acc,
      params["gcn_W"][:128], params["gcn_W"][128:], row(params["gcn_b"]),
      row(params["gcn_g"]), row(params["gcn_beta"]),
      nWp, nbp, ngp, nbep,
      W1[0:128], W1bp, W1[137:265], W1ep, row(params["b1"]))

    gather = pl.kernel(
        _sc_gather_body,
        out_type=jax.ShapeDtypeStruct((SEE, 128), f32),
        mesh=mesh,
        compiler_params=pltpu.CompilerParams(needs_layout_passes=False),
        scratch_types=[
            pltpu.VMEM((NCHK2, CH), jnp.int32),
            pltpu.VMEM((NCHK2, CH), jnp.int32),
            pltpu.VMEM((CH, 128), f32),
            pltpu.VMEM((CH, 128), f32),
            pltpu.VMEM((CH, 128), f32),
            pltpu.VMEM((CH, 128), f32),
            pltpu.SemaphoreType.DMA,
            pltpu.SemaphoreType.DMA,
            pltpu.SemaphoreType.DMA,
            pltpu.SemaphoreType.DMA,
            pltpu.SemaphoreType.DMA,
            pltpu.SemaphoreType.DMA,
        ],
    )
    srcp5 = edge_index[0].reshape(NSLC, NW, NCHK2, CH)
    dstp5 = edge_index[1].reshape(NSLC, NW, NCHK2, CH)
    eaT = edge_attr.T
    w1c = W1[133:137]
    w1ce, w1co = w1c[:, 0::2], w1c[:, 1::2]
    lnp = jnp.concatenate([params["ln1_g"][0::2], params["ln1_g"][1::2]])
    lbp = jnp.concatenate([params["ln1_b"][0::2], params["ln1_b"][1::2]])
    w2perm = jnp.concatenate([params["W2"][0::2], params["W2"][1::2]], axis=0)
    w2t8 = jnp.zeros((8, DH), f32).at[:2].set(w2perm.T)
    b2c = params["b2"].reshape(2, 1)
    edge_tail = pl.pallas_call(
        _edge_body,
        grid=(NEB2,),
        in_specs=[pl.BlockSpec((EBLK, 128), lambda i: (i, 0)),
                  pl.BlockSpec((4, EBLK), lambda i: (0, i)),
                  _full((4, 128)), _full((4, 128)), _full((1, DH)),
                  _full((1, DH)), _full((8, DH)), _full((2, 1))],
        out_specs=pl.BlockSpec((2, EBLK), lambda i: (0, i)),
        out_shape=jax.ShapeDtypeStruct((2, SEE), f32),
    )
    tabAp = lax.bitcast_convert_type(tabA.reshape(NP, 128, 2), f32)
    tabBp = lax.bitcast_convert_type(tabB.reshape(NP, 128, 2), f32)
    eTs = []
    for sl in range(NSLC):
        tsum = gather(tabAp, tabBp, srcp5[sl], dstp5[sl])
        eTs.append(edge_tail(
            tsum, lax.dynamic_slice_in_dim(eaT, sl * SEE, SEE, axis=1),
            w1ce, w1co, row(lnp), row(lbp), w2t8, b2c))
    eT = jnp.concatenate(eTs, axis=1)

    n = nfull[:NN, :5]
    return (n, eT.T)


# speedup vs baseline: 9.4689x; 1.2361x over previous
import functools

import jax
import jax.numpy as jnp
from jax import lax
from jax.experimental import pallas as pl
from jax.experimental.pallas import tpu as pltpu
from jax.experimental.pallas import tpu_sc as plsc

NN = 10000
NP = 10240
RW = 136
EE = 320000
NC, NS = 2, 16
NW = NC * NS
EPT = EE // NW
CH = 80
NCHK = EPT // CH
RPS = NP // NS
DH = 256
NBLK = 8
BLK = NP // NBLK
EBLK = 6400
EPS = 1e-5
NSLC = 5
SEE = EE // NSLC
EPTS = SEE // NW
NCHK2 = EPTS // CH
NEB2 = SEE // EBLK


def _round_pack(lo, hi):
    ul = lax.bitcast_convert_type(lo, jnp.int32)
    uh = lax.bitcast_convert_type(hi, jnp.int32)
    rl = ul + jnp.int32(0x7FFF) + (lax.shift_right_logical(ul, 16) & 1)
    lo16 = lax.shift_right_logical(rl, 16)
    rh = (uh + jnp.int32(0x7FFF) + (lax.shift_right_logical(uh, 16) & 1)) & jnp.int32(-65536)
    return lax.bitcast_convert_type(rh | lo16, jnp.float32)


def _ln(x, g, b):
    m = jnp.mean(x, axis=1, keepdims=True)
    v = jnp.mean((x - m) ** 2, axis=1, keepdims=True)
    return (x - m) * lax.rsqrt(v + EPS) * g + b


def _proj_body(h_ref, w0, b0, g0, be0, w1, b1, g1, be1, out_ref):
    h = h_ref[...]
    z0 = jnp.maximum(_ln(h[:, :64] @ w0[...] + b0[...], g0[...], be0[...]), 0.0)
    z1 = jnp.maximum(_ln(h[:, 64:] @ w1[...] + b1[...], g1[...], be1[...]), 0.0)
    out_ref[:, 0:64] = z0
    out_ref[:, 64:128] = z1
    col = lax.broadcasted_iota(jnp.int32, (BLK, 8), 1)
    out_ref[:, 128:136] = jnp.where(col == 0, 1.0, 0.0)


def _sc_scatter_body(aug_hbm, srcp_hbm, dstp_hbm, zr_hbm, acc_hbm,
                     acc_sh, src_v, dst_v, rows_v, rows_v1, sem, sem1):
    c = lax.axis_index("c")
    s = lax.axis_index("s")
    wid = s * NC + c
    pltpu.sync_copy(zr_hbm, acc_sh.at[pl.ds(s * RPS, RPS)])
    plsc.subcore_barrier()
    pltpu.sync_copy(srcp_hbm.at[wid], src_v)
    pltpu.sync_copy(dstp_hbm.at[wid], dst_v)

    def issue(j, rv, sg):
        pltpu.async_copy(aug_hbm.at[src_v.at[j]], rv, sg)

    def wait_gather(j, rv, sg):
        pltpu.make_async_copy(aug_hbm.at[src_v.at[j]], rv, sg).wait()

    issue(0, rows_v, sem)

    def chunk2(jj, carry):
        j0 = 2 * jj
        j1 = j0 + 1

        @pl.when(j1 < NCHK)
        def _():
            issue(j1, rows_v1, sem1)
        wait_gather(j0, rows_v, sem)
        pltpu.sync_copy(rows_v, acc_sh.at[dst_v.at[j0]], add=True)

        @pl.when(j1 < NCHK)
        def _():
            @pl.when(j1 + 1 < NCHK)
            def _():
                issue(j1 + 1, rows_v, sem)
            wait_gather(j1, rows_v1, sem1)
            pltpu.sync_copy(rows_v1, acc_sh.at[dst_v.at[j1]], add=True)
        return carry

    lax.fori_loop(0, (NCHK + 1) // 2, chunk2, 0)
    plsc.subcore_barrier()
    pltpu.sync_copy(acc_sh.at[pl.ds(s * RPS, RPS)],
                    acc_hbm.at[c, pl.ds(s * RPS, RPS)])


def _mid_body(aug_ref, acc_ref, gwa, gwb, gb, gg, gbe,
              nwp, nbp, ngp, nbep, w1a, w1bp, w1d, w1ep, b1,
              n_ref, a_ref, b_ref):
    hp = aug_ref[:, :128]
    ssum = acc_ref[0] + acc_ref[1]
    ah = ssum[:, :128]
    deg = ssum[:, 128:129]
    nrm = jnp.where(deg > 0, 1.0 / deg, 0.0)
    pre = hp @ gwa[...] + (ah * nrm) @ gwb[...] + gb[...]
    hg = jnp.maximum(_ln(pre, gg[...], gbe[...]), 0.0)
    nh = hg @ nwp[...] + nbp[...]
    cmask = (lax.broadcasted_iota(jnp.int32, (1, 128), 1) < 5).astype(jnp.float32)
    nm = jnp.sum(nh * cmask, axis=1, keepdims=True) * (1.0 / 5.0)
    d = (nh - nm) * cmask
    nv = jnp.sum(d * d, axis=1, keepdims=True) * (1.0 / 5.0)
    nf = d * lax.rsqrt(nv + EPS) * ngp[...] + nbep[...]
    n_ref[...] = nf
    ta = hg @ w1a[...] + nf @ w1bp[...] + b1[...]
    tb = hg @ w1d[...] + nf @ w1ep[...]
    a_ref[...] = _round_pack(ta[:, :128], ta[:, 128:])
    b_ref[...] = _round_pack(tb[:, :128], tb[:, 128:])


def _sc_gather_body(a_hbm, b_hbm, srcp_hbm, dstp_hbm, out_hbm,
                    src_v, dst_v, bufa0, bufb0, bufa1, bufb1,
                    sema0, semb0, sema1, semb1, semw0, semw1):
    c = lax.axis_index("c")
    s = lax.axis_index("s")
    wid = s * NC + c
    pltpu.sync_copy(srcp_hbm.at[wid], src_v)
    pltpu.sync_copy(dstp_hbm.at[wid], dst_v)
    base = wid * EPTS

    def issue(j, ba, bb, sa, sb):
        pltpu.async_copy(a_hbm.at[src_v.at[j]], ba, sa)
        pltpu.async_copy(b_hbm.at[dst_v.at[j]], bb, sb)

    def wait_gather(j, ba, bb, sa, sb):
        pltpu.make_async_copy(a_hbm.at[src_v.at[j]], ba, sa).wait()
        pltpu.make_async_copy(b_hbm.at[dst_v.at[j]], bb, sb).wait()

    def add_rows(ba, bb):
        def row(r2, cc):
            for u in range(2):
                r = r2 * 2 + u
                for k in range(128 // 16):
                    sl = pl.ds(k * 16, 16)
                    s = (plsc.bitcast(ba[r, sl], jnp.bfloat16) +
                         plsc.bitcast(bb[r, sl], jnp.bfloat16))
                    ba[r, sl] = plsc.bitcast(s, jnp.float32)
            return cc
        lax.fori_loop(0, CH // 2, row, 0)

    def drain_wb(j, ba, sw):
        pltpu.make_async_copy(ba, out_hbm.at[pl.ds(base + j * CH, CH)], sw).wait()

    issue(0, bufa0, bufb0, sema0, semb0)

    def outer(jj, carry):
        j0 = 2 * jj
        j1 = j0 + 1

        @pl.when(j1 < NCHK2)
        def _():
            issue(j1, bufa1, bufb1, sema1, semb1)
        wait_gather(j0, bufa0, bufb0, sema0, semb0)
        add_rows(bufa0, bufb0)
        pltpu.async_copy(bufa0, out_hbm.at[pl.ds(base + j0 * CH, CH)], semw0)

        @pl.when(j1 < NCHK2)
        def _():
            @pl.when(j1 + 1 < NCHK2)
            def _():
                drain_wb(j0, bufa0, semw0)
                issue(j1 + 1, bufa0, bufb0, sema0, semb0)
            wait_gather(j1, bufa1, bufb1, sema1, semb1)
            add_rows(bufa1, bufb1)
            pltpu.sync_copy(bufa1, out_hbm.at[pl.ds(base + j1 * CH, CH)])

        @pl.when(j1 >= NCHK2)
        def _():
            drain_wb(j0, bufa0, semw0)
        return carry

    lax.fori_loop(0, (NCHK2 + 1) // 2, outer, 0)


def _edge_body(t_ref, eat_ref, w1ce, w1co, lg, lb, w2t8, b2c, out_ref):
    wi = lax.bitcast_convert_type(t_ref[...], jnp.int32)
    te = lax.bitcast_convert_type(wi << 16, jnp.float32)
    to = lax.bitcast_convert_type(wi & jnp.int32(-65536), jnp.float32)
    eat = eat_ref[...]
    x0 = te + lax.dot_general(eat, w1ce[...], (((0,), (0,)), ((), ())))
    x1 = to + lax.dot_general(eat, w1co[...], (((0,), (0,)), ((), ())))
    m = (jnp.sum(x0, axis=1, keepdims=True) +
         jnp.sum(x1, axis=1, keepdims=True)) * (1.0 / DH)
    d0 = x0 - m
    d1 = x1 - m
    v = (jnp.sum(d0 * d0, axis=1, keepdims=True) +
         jnp.sum(d1 * d1, axis=1, keepdims=True)) * (1.0 / DH)
    rs = lax.rsqrt(v + EPS)
    lgv = lg[...]
    lbv = lb[...]
    y0 = jnp.maximum(d0 * rs * lgv[:, :128] + lbv[:, :128], 0.0)
    y1 = jnp.maximum(d1 * rs * lgv[:, 128:] + lbv[:, 128:], 0.0)
    w2v = w2t8[...]
    e8 = (lax.dot_general(w2v[:, :128], y0, (((1,), (1,)), ((), ()))) +
          lax.dot_general(w2v[:, 128:], y1, (((1,), (1,)), ((), ()))))
    out_ref[...] = e8[:2, :] + b2c[...]


def _full(shape):
    return pl.BlockSpec(shape, lambda i: tuple(0 for _ in shape))


def kernel(h, edge_index, edge_attr, params):
    f32 = jnp.float32
    hpad = jnp.zeros((NP, 128), f32).at[:NN].set(h)
    srcp = edge_index[0].reshape(NW, NCHK, CH)
    dstp = edge_index[1].reshape(NW, NCHK, CH)
    zr = jnp.zeros((RPS, RW), f32)

    p0, p1 = params["proj"][0], params["proj"][1]
    row = lambda v: v.reshape(1, -1)

    aug = pl.pallas_call(
        _proj_body,
        grid=(NBLK,),
        in_specs=[pl.BlockSpec((BLK, 128), lambda i: (i, 0)),
                  _full((64, 64)), _full((1, 64)), _full((1, 64)), _full((1, 64)),
                  _full((64, 64)), _full((1, 64)), _full((1, 64)), _full((1, 64))],
        out_specs=pl.BlockSpec((BLK, RW), lambda i: (i, 0)),
        out_shape=jax.ShapeDtypeStruct((NP, RW), f32),
    )(hpad, p0["W"], row(p0["b"]), row(p0["g"]), row(p0["beta"]),
      p1["W"], row(p1["b"]), row(p1["g"]), row(p1["beta"]))

    mesh = plsc.VectorSubcoreMesh(core_axis_name="c", subcore_axis_name="s",
                                  num_cores=NC, num_subcores=NS)
    scatter = pl.kernel(
        _sc_scatter_body,
        out_type=jax.ShapeDtypeStruct((NC, NP, RW), f32),
        mesh=mesh,
        compiler_params=pltpu.CompilerParams(use_tc_tiling_on_sc=False),
        scratch_types=[
            pltpu.VMEM_SHARED((NP, RW), f32),
            pltpu.VMEM((NCHK, CH), jnp.int32),
            pltpu.VMEM((NCHK, CH), jnp.int32),
            pltpu.VMEM((CH, RW), f32),
            pltpu.VMEM((CH, RW), f32),
            pltpu.SemaphoreType.DMA,
            pltpu.SemaphoreType.DMA,
        ],
    )
    acc = scatter(aug, srcp, dstp, zr)

    W1 = params["W1"]
    nWp = jnp.zeros((128, 128), f32).at[:, :5].set(params["node_W"])
    nbp = jnp.zeros((1, 128), f32).at[0, :5].set(params["node_b"])
    ngp = jnp.zeros((1, 128), f32).at[0, :5].set(params["node_g"])
    nbep = jnp.zeros((1, 128), f32).at[0, :5].set(params["node_beta"])
    W1bp = jnp.zeros((128, DH), f32).at[:5].set(W1[128:133])
    W1ep = jnp.zeros((128, DH), f32).at[:5].set(W1[265:270])
    nfull, tabA, tabB = pl.pallas_call(
        _mid_body,
        grid=(NBLK,),
        in_specs=[pl.BlockSpec((BLK, RW), lambda i: (i, 0)),
                  pl.BlockSpec((NC, BLK, RW), lambda i: (0, i, 0)),
                  _full((128, 128)), _full((128, 128)), _full((1, 128)),
                  _full((1, 128)), _full((1, 128)),
                  _full((128, 128)), _full((1, 128)), _full((1, 128)), _full((1, 128)),
                  _full((128, DH)), _full((128, DH)), _full((128, DH)),
                  _full((128, DH)), _full((1, DH))],
        out_specs=[pl.BlockSpec((BLK, 128), lambda i: (i, 0)),
                   pl.BlockSpec((BLK, 128), lambda i: (i, 0)),
                   pl.BlockSpec((BLK, 128), lambda i: (i, 0))],
        out_shape=[jax.ShapeDtypeStruct((NP, 128), f32),
                   jax.ShapeDtypeStruct((NP, 128), f32),
                   jax.ShapeDtypeStruct((NP, 128), f32)],
    )(aug, acc,
      params["gcn_W"][:128], params["gcn_W"][128:], row(params["gcn_b"]),
      row(params["gcn_g"]), row(params["gcn_beta"]),
      nWp, nbp, ngp, nbep,
      W1[0:128], W1bp, W1[137:265], W1ep, row(params["b1"]))

    gather = pl.kernel(
        _sc_gather_body,
        out_type=jax.ShapeDtypeStruct((SEE, 128), f32),
        mesh=mesh,
        compiler_params=pltpu.CompilerParams(needs_layout_passes=False),
        scratch_types=[
            pltpu.VMEM((NCHK2, CH), jnp.int32),
            pltpu.VMEM((NCHK2, CH), jnp.int32),
            pltpu.VMEM((CH, 128), f32),
            pltpu.VMEM((CH, 128), f32),
            pltpu.VMEM((CH, 128), f32),
            pltpu.VMEM((CH, 128), f32),
            pltpu.SemaphoreType.DMA,
            pltpu.SemaphoreType.DMA,
            pltpu.SemaphoreType.DMA,
            pltpu.SemaphoreType.DMA,
            pltpu.SemaphoreType.DMA,
            pltpu.SemaphoreType.DMA,
        ],
    )
    srcp5 = edge_index[0].reshape(NSLC, NW, NCHK2, CH)
    dstp5 = edge_index[1].reshape(NSLC, NW, NCHK2, CH)
    eaT = edge_attr.T
    w1c = W1[133:137]
    w1ce, w1co = w1c[:, :128], w1c[:, 128:]
    w2t8 = jnp.zeros((8, DH), f32).at[:2].set(params["W2"].T)
    b2c = params["b2"].reshape(2, 1)
    edge_tail = pl.pallas_call(
        _edge_body,
        grid=(NEB2,),
        in_specs=[pl.BlockSpec((EBLK, 128), lambda i: (i, 0)),
                  pl.BlockSpec((4, EBLK), lambda i: (0, i)),
                  _full((4, 128)), _full((4, 128)), _full((1, DH)),
                  _full((1, DH)), _full((8, DH)), _full((2, 1))],
        out_specs=pl.BlockSpec((2, EBLK), lambda i: (0, i)),
        out_shape=jax.ShapeDtypeStruct((2, SEE), f32),
    )
    eTs = []
    for sl in range(NSLC):
        tsum = gather(tabA, tabB, srcp5[sl], dstp5[sl])
        eTs.append(edge_tail(
            tsum, lax.dynamic_slice_in_dim(eaT, sl * SEE, SEE, axis=1),
            w1ce, w1co, row(params["ln1_g"]), row(params["ln1_b"]), w2t8, b2c))
    eT = jnp.concatenate(eTs, axis=1)

    n = nfull[:NN, :5]
    return (n, eT.T)


# speedup vs baseline: 9.4800x; 1.0012x over previous
import functools

import jax
import jax.numpy as jnp
from jax import lax
from jax.experimental import pallas as pl
from jax.experimental.pallas import tpu as pltpu
from jax.experimental.pallas import tpu_sc as plsc

NN = 10000
NP = 10240
RW = 136
EE = 320000
NC, NS = 2, 16
NW = NC * NS
EPT = EE // NW
CH = 80
NCHK = EPT // CH
RPS = NP // NS
DH = 256
NBLK = 8
BLK = NP // NBLK
EBLK = 6400
EPS = 1e-5
NSLC = 5
SEE = EE // NSLC
EPTS = SEE // NW
NCHK2 = EPTS // CH
NEB2 = SEE // EBLK


def _round_pack(lo, hi):
    ul = lax.bitcast_convert_type(lo, jnp.int32)
    uh = lax.bitcast_convert_type(hi, jnp.int32)
    rl = ul + jnp.int32(0x7FFF) + (lax.shift_right_logical(ul, 16) & 1)
    lo16 = lax.shift_right_logical(rl, 16)
    rh = (uh + jnp.int32(0x7FFF) + (lax.shift_right_logical(uh, 16) & 1)) & jnp.int32(-65536)
    return lax.bitcast_convert_type(rh | lo16, jnp.float32)


def _ln(x, g, b):
    m = jnp.mean(x, axis=1, keepdims=True)
    v = jnp.mean((x - m) ** 2, axis=1, keepdims=True)
    return (x - m) * lax.rsqrt(v + EPS) * g + b


def _proj_body(h_ref, w0, b0, g0, be0, w1, b1, g1, be1, out_ref):
    h = h_ref[...]
    z0 = jnp.maximum(_ln(h[:, :64] @ w0[...] + b0[...], g0[...], be0[...]), 0.0)
    z1 = jnp.maximum(_ln(h[:, 64:] @ w1[...] + b1[...], g1[...], be1[...]), 0.0)
    out_ref[:, 0:64] = z0
    out_ref[:, 64:128] = z1
    col = lax.broadcasted_iota(jnp.int32, (BLK, 8), 1)
    out_ref[:, 128:136] = jnp.where(col == 0, 1.0, 0.0)


def _sc_scatter_body(aug_hbm, srcp_hbm, dstp_hbm, zr_hbm, acc_hbm,
                     acc_sh, src_v, dst_v, rows_v, rows_v1, sem, sem1):
    c = lax.axis_index("c")
    s = lax.axis_index("s")
    wid = s * NC + c
    def issue(j, rv, sg):
        pltpu.async_copy(aug_hbm.at[src_v.at[j]], rv, sg)

    def wait_gather(j, rv, sg):
        pltpu.make_async_copy(aug_hbm.at[src_v.at[j]], rv, sg).wait()

    pltpu.sync_copy(srcp_hbm.at[wid], src_v)
    pltpu.sync_copy(dstp_hbm.at[wid], dst_v)
    issue(0, rows_v, sem)
    issue(1, rows_v1, sem1)
    pltpu.sync_copy(zr_hbm, acc_sh.at[pl.ds(s * RPS, RPS)])
    plsc.subcore_barrier()

    def chunk2(jj, carry):
        j0 = 2 * jj
        j1 = j0 + 1
        wait_gather(j0, rows_v, sem)
        pltpu.sync_copy(rows_v, acc_sh.at[dst_v.at[j0]], add=True)

        @pl.when(j0 + 2 < NCHK)
        def _():
            issue(j0 + 2, rows_v, sem)

        @pl.when(j1 < NCHK)
        def _():
            wait_gather(j1, rows_v1, sem1)
            pltpu.sync_copy(rows_v1, acc_sh.at[dst_v.at[j1]], add=True)

            @pl.when(j1 + 2 < NCHK)
            def _():
                issue(j1 + 2, rows_v1, sem1)
        return carry

    lax.fori_loop(0, (NCHK + 1) // 2, chunk2, 0)
    plsc.subcore_barrier()
    pltpu.sync_copy(acc_sh.at[pl.ds(s * RPS, RPS)],
                    acc_hbm.at[c, pl.ds(s * RPS, RPS)])


def _mid_body(aug_ref, acc_ref, gwa, gwb, gb, gg, gbe,
              nwp, nbp, ngp, nbep, w1a, w1bp, w1d, w1ep, b1,
              n_ref, a_ref, b_ref):
    hp = aug_ref[:, :128]
    ssum = acc_ref[0] + acc_ref[1]
    ah = ssum[:, :128]
    deg = ssum[:, 128:129]
    nrm = jnp.where(deg > 0, 1.0 / deg, 0.0)
    pre = hp @ gwa[...] + (ah * nrm) @ gwb[...] + gb[...]
    hg = jnp.maximum(_ln(pre, gg[...], gbe[...]), 0.0)
    nh = hg @ nwp[...] + nbp[...]
    cmask = (lax.broadcasted_iota(jnp.int32, (1, 128), 1) < 5).astype(jnp.float32)
    nm = jnp.sum(nh * cmask, axis=1, keepdims=True) * (1.0 / 5.0)
    d = (nh - nm) * cmask
    nv = jnp.sum(d * d, axis=1, keepdims=True) * (1.0 / 5.0)
    nf = d * lax.rsqrt(nv + EPS) * ngp[...] + nbep[...]
    n_ref[...] = nf
    ta = hg @ w1a[...] + nf @ w1bp[...] + b1[...]
    tb = hg @ w1d[...] + nf @ w1ep[...]
    a_ref[...] = _round_pack(ta[:, :128], ta[:, 128:])
    b_ref[...] = _round_pack(tb[:, :128], tb[:, 128:])


def _sc_gather_body(a_hbm, b_hbm, srcp_hbm, dstp_hbm, out_hbm,
                    src_v, dst_v, bufa0, bufb0, bufa1, bufb1,
                    sema0, semb0, sema1, semb1, semw0, semw1):
    c = lax.axis_index("c")
    s = lax.axis_index("s")
    wid = s * NC + c
    pltpu.sync_copy(srcp_hbm.at[wid], src_v)
    pltpu.sync_copy(dstp_hbm.at[wid], dst_v)
    base = wid * EPTS

    def issue(j, ba, bb, sa, sb):
        pltpu.async_copy(a_hbm.at[src_v.at[j]], ba, sa)
        pltpu.async_copy(b_hbm.at[dst_v.at[j]], bb, sb)

    def wait_gather(j, ba, bb, sa, sb):
        pltpu.make_async_copy(a_hbm.at[src_v.at[j]], ba, sa).wait()
        pltpu.make_async_copy(b_hbm.at[dst_v.at[j]], bb, sb).wait()

    def add_rows(ba, bb):
        def row(r2, cc):
            for u in range(2):
                r = r2 * 2 + u
                for k in range(128 // 16):
                    sl = pl.ds(k * 16, 16)
                    s = (plsc.bitcast(ba[r, sl], jnp.bfloat16) +
                         plsc.bitcast(bb[r, sl], jnp.bfloat16))
                    ba[r, sl] = plsc.bitcast(s, jnp.float32)
            return cc
        lax.fori_loop(0, CH // 2, row, 0)

    def drain_wb(j, ba, sw):
        pltpu.make_async_copy(ba, out_hbm.at[pl.ds(base + j * CH, CH)], sw).wait()

    issue(0, bufa0, bufb0, sema0, semb0)

    def outer(jj, carry):
        j0 = 2 * jj
        j1 = j0 + 1

        @pl.when(j1 < NCHK2)
        def _():
            issue(j1, bufa1, bufb1, sema1, semb1)
        wait_gather(j0, bufa0, bufb0, sema0, semb0)
        add_rows(bufa0, bufb0)
        pltpu.async_copy(bufa0, out_hbm.at[pl.ds(base + j0 * CH, CH)], semw0)

        @pl.when(j1 < NCHK2)
        def _():
            @pl.when(j1 + 1 < NCHK2)
            def _():
                drain_wb(j0, bufa0, semw0)
                issue(j1 + 1, bufa0, bufb0, sema0, semb0)
            wait_gather(j1, bufa1, bufb1, sema1, semb1)
            add_rows(bufa1, bufb1)
            pltpu.sync_copy(bufa1, out_hbm.at[pl.ds(base + j1 * CH, CH)])

        @pl.when(j1 >= NCHK2)
        def _():
            drain_wb(j0, bufa0, semw0)
        return carry

    lax.fori_loop(0, (NCHK2 + 1) // 2, outer, 0)


def _edge_body(t_ref, eat_ref, w1ce, w1co, lg, lb, w2t8, b2c, out_ref):
    wi = lax.bitcast_convert_type(t_ref[...], jnp.int32)
    te = lax.bitcast_convert_type(wi << 16, jnp.float32)
    to = lax.bitcast_convert_type(wi & jnp.int32(-65536), jnp.float32)
    eat = eat_ref[...]
    x0 = te + lax.dot_general(eat, w1ce[...], (((0,), (0,)), ((), ())))
    x1 = to + lax.dot_general(eat, w1co[...], (((0,), (0,)), ((), ())))
    m = (jnp.sum(x0, axis=1, keepdims=True) +
         jnp.sum(x1, axis=1, keepdims=True)) * (1.0 / DH)
    d0 = x0 - m
    d1 = x1 - m
    v = (jnp.sum(d0 * d0, axis=1, keepdims=True) +
         jnp.sum(d1 * d1, axis=1, keepdims=True)) * (1.0 / DH)
    rs = lax.rsqrt(v + EPS)
    lgv = lg[...]
    lbv = lb[...]
    y0 = jnp.maximum(d0 * rs * lgv[:, :128] + lbv[:, :128], 0.0)
    y1 = jnp.maximum(d1 * rs * lgv[:, 128:] + lbv[:, 128:], 0.0)
    w2v = w2t8[...]
    e8 = (lax.dot_general(w2v[:, :128], y0, (((1,), (1,)), ((), ()))) +
          lax.dot_general(w2v[:, 128:], y1, (((1,), (1,)), ((), ()))))
    out_ref[...] = e8[:2, :] + b2c[...]


def _full(shape):
    return pl.BlockSpec(shape, lambda i: tuple(0 for _ in shape))


def kernel(h, edge_index, edge_attr, params):
    f32 = jnp.float32
    hpad = jnp.zeros((NP, 128), f32).at[:NN].set(h)
    srcp = edge_index[0].reshape(NW, NCHK, CH)
    dstp = edge_index[1].reshape(NW, NCHK, CH)
    zr = jnp.zeros((RPS, RW), f32)

    p0, p1 = params["proj"][0], params["proj"][1]
    row = lambda v: v.reshape(1, -1)

    aug = pl.pallas_call(
        _proj_body,
        grid=(NBLK,),
        in_specs=[pl.BlockSpec((BLK, 128), lambda i: (i, 0)),
                  _full((64, 64)), _full((1, 64)), _full((1, 64)), _full((1, 64)),
                  _full((64, 64)), _full((1, 64)), _full((1, 64)), _full((1, 64))],
        out_specs=pl.BlockSpec((BLK, RW), lambda i: (i, 0)),
        out_shape=jax.ShapeDtypeStruct((NP, RW), f32),
    )(hpad, p0["W"], row(p0["b"]), row(p0["g"]), row(p0["beta"]),
      p1["W"], row(p1["b"]), row(p1["g"]), row(p1["beta"]))

    mesh = plsc.VectorSubcoreMesh(core_axis_name="c", subcore_axis_name="s",
                                  num_cores=NC, num_subcores=NS)
    scatter = pl.kernel(
        _sc_scatter_body,
        out_type=jax.ShapeDtypeStruct((NC, NP, RW), f32),
        mesh=mesh,
        compiler_params=pltpu.CompilerParams(use_tc_tiling_on_sc=False),
        scratch_types=[
            pltpu.VMEM_SHARED((NP, RW), f32),
            pltpu.VMEM((NCHK, CH), jnp.int32),
            pltpu.VMEM((NCHK, CH), jnp.int32),
            pltpu.VMEM((CH, RW), f32),
            pltpu.VMEM((CH, RW), f32),
            pltpu.SemaphoreType.DMA,
            pltpu.SemaphoreType.DMA,
        ],
    )
    acc = scatter(aug, srcp, dstp, zr)

    W1 = params["W1"]
    nWp = jnp.zeros((128, 128), f32).at[:, :5].set(params["node_W"])
    nbp = jnp.zeros((1, 128), f32).at[0, :5].set(params["node_b"])
    ngp = jnp.zeros((1, 128), f32).at[0, :5].set(params["node_g"])
    nbep = jnp.zeros((1, 128), f32).at[0, :5].set(params["node_beta"])
    W1bp = jnp.zeros((128, DH), f32).at[:5].set(W1[128:133])
    W1ep = jnp.zeros((128, DH), f32).at[:5].set(W1[265:270])
    nfull, tabA, tabB = pl.pallas_call(
        _mid_body,
        grid=(NBLK,),
        in_specs=[pl.BlockSpec((BLK, RW), lambda i: (i, 0)),
                  pl.BlockSpec((NC, BLK, RW), lambda i: (0, i, 0)),
                  _full((128, 128)), _full((128, 128)), _full((1, 128)),
                  _full((1, 128)), _full((1, 128)),
                  _full((128, 128)), _full((1, 128)), _full((1, 128)), _full((1, 128)),
                  _full((128, DH)), _full((128, DH)), _full((128, DH)),
                  _full((128, DH)), _full((1, DH))],
        out_specs=[pl.BlockSpec((BLK, 128), lambda i: (i, 0)),
                   pl.BlockSpec((BLK, 128), lambda i: (i, 0)),
                   pl.BlockSpec((BLK, 128), lambda i: (i, 0))],
        out_shape=[jax.ShapeDtypeStruct((NP, 128), f32),
                   jax.ShapeDtypeStruct((NP, 128), f32),
                   jax.ShapeDtypeStruct((NP, 128), f32)],
    )(aug, acc,
      params["gcn_W"][:128], params["gcn_W"][128:], row(params["gcn_b"]),
      row(params["gcn_g"]), row(params["gcn_beta"]),
      nWp, nbp, ngp, nbep,
      W1[0:128], W1bp, W1[137:265], W1ep, row(params["b1"]))

    gather = pl.kernel(
        _sc_gather_body,
        out_type=jax.ShapeDtypeStruct((SEE, 128), f32),
        mesh=mesh,
        compiler_params=pltpu.CompilerParams(needs_layout_passes=False),
        scratch_types=[
            pltpu.VMEM((NCHK2, CH), jnp.int32),
            pltpu.VMEM((NCHK2, CH), jnp.int32),
            pltpu.VMEM((CH, 128), f32),
            pltpu.VMEM((CH, 128), f32),
            pltpu.VMEM((CH, 128), f32),
            pltpu.VMEM((CH, 128), f32),
            pltpu.SemaphoreType.DMA,
            pltpu.SemaphoreType.DMA,
            pltpu.SemaphoreType.DMA,
            pltpu.SemaphoreType.DMA,
            pltpu.SemaphoreType.DMA,
            pltpu.SemaphoreType.DMA,
        ],
    )
    srcp5 = edge_index[0].reshape(NSLC, NW, NCHK2, CH)
    dstp5 = edge_index[1].reshape(NSLC, NW, NCHK2, CH)
    eaT = edge_attr.T
    w1c = W1[133:137]
    w1ce, w1co = w1c[:, :128], w1c[:, 128:]
    w2t8 = jnp.zeros((8, DH), f32).at[:2].set(params["W2"].T)
    b2c = params["b2"].reshape(2, 1)
    edge_tail = pl.pallas_call(
        _edge_body,
        grid=(NEB2,),
        in_specs=[pl.BlockSpec((EBLK, 128), lambda i: (i, 0)),
                  pl.BlockSpec((4, EBLK), lambda i: (0, i)),
                  _full((4, 128)), _full((4, 128)), _full((1, DH)),
                  _full((1, DH)), _full((8, DH)), _full((2, 1))],
        out_specs=pl.BlockSpec((2, EBLK), lambda i: (0, i)),
        out_shape=jax.ShapeDtypeStruct((2, SEE), f32),
    )
    eTs = []
    for sl in range(NSLC):
        tsum = gather(tabA, tabB, srcp5[sl], dstp5[sl])
        eTs.append(edge_tail(
            tsum, lax.dynamic_slice_in_dim(eaT, sl * SEE, SEE, axis=1),
            w1ce, w1co, row(params["ln1_g"]), row(params["ln1_b"]), w2t8, b2c))
    eT = jnp.concatenate(eTs, axis=1)

    n = nfull[:NN, :5]
    return (n, eT.T)
